# Initial kernel scaffold; baseline (speedup 1.0000x reference)
import jax
import jax.numpy as jnp
from jax.experimental import pallas as pl


def kernel(x, edge_index, batch, W_emb, b_emb, W1, att_src1, att_dst1, b1, W2, att_src2, att_dst2, b2, gamma1, beta1, gamma2, beta2, W_out, b_out):
    raise NotImplementedError("write your pallas kernel here")


# speedup vs baseline: 29.1902x; 29.1902x over previous
import functools

import jax
import jax.numpy as jnp
from jax import lax
from jax.experimental import pallas as pl
from jax.experimental.pallas import tpu as pltpu
from jax.experimental.pallas import tpu_sc as plsc

RB = 1000
CE = 1024
SUB = 128
NEG_INF = float("-inf")


def _elu(x):
  return jnp.where(x > 0, x, jnp.exp(jnp.minimum(x, 0.0)) - 1.0)


def _prep_kernel(embed, h_r, wemb_r, bemb_r, w_r, a_r, b_r,
                 hw_o, as_o, ad_o, m_o):
  i = pl.program_id(0)
  h = h_r[...]
  if embed:
    h = _elu(h @ wemb_r[...] + bemb_r[...])
  hw = h @ w_r[...]
  hw_o[...] = hw
  asb = hw @ a_r[...]
  adb = hw @ b_r[...]
  as_o[...] = asb
  ad_o[...] = adb

  @pl.when(i == 0)
  def _():
    m_o[...] = jnp.full_like(m_o[...], NEG_INF)

  m_o[...] = jnp.maximum(m_o[...], jnp.max(asb, axis=0, keepdims=True))


def _prep_call(embed, h_in, wemb, bemb, w, a, b, heads):
  n = h_in.shape[0]
  nb = n // RB
  cin = h_in.shape[1]
  full = lambda shp: pl.BlockSpec(shp, lambda i: (0, 0))
  return pl.pallas_call(
      functools.partial(_prep_kernel, embed),
      grid=(nb,),
      in_specs=[
          pl.BlockSpec((RB, cin), lambda i: (i, 0)),
          full(wemb.shape), full(bemb.shape), full(w.shape),
          full(a.shape), full(b.shape),
      ],
      out_specs=[
          pl.BlockSpec((RB, 64), lambda i: (i, 0)),
          pl.BlockSpec((RB, heads), lambda i: (i, 0)),
          pl.BlockSpec((RB, heads), lambda i: (i, 0)),
          pl.BlockSpec((1, heads), lambda i: (0, 0)),
      ],
      out_shape=[
          jax.ShapeDtypeStruct((n, 64), jnp.float32),
          jax.ShapeDtypeStruct((n, heads), jnp.float32),
          jax.ShapeDtypeStruct((n, heads), jnp.float32),
          jax.ShapeDtypeStruct((1, heads), jnp.float32),
      ],
  )(h_in, wemb, bemb, w, a, b)


def _post_kernel(heads, n, num_r, den_r, as_r, ad_r, m_r, hw_r, b_r,
                 conv_o, ssum_o, ssq_o):
  i = pl.program_id(0)
  c = 64 // heads
  asb = as_r[...]
  adb = ad_r[...]
  m = m_r[...]
  t = asb + adb
  al = jnp.maximum(t, 0.2 * t)
  t2 = m + adb
  d2 = jnp.maximum(t2, 0.2 * t2)
  sex = jnp.exp(al - d2)
  den = den_r[...] + sex
  hw = hw_r[...]
  num = num_r[...]
  parts = []
  for h in range(heads):
    nh = num[:, h * c:(h + 1) * c] + sex[:, h:h + 1] * hw[:, h * c:(h + 1) * c]
    parts.append(nh / (den[:, h:h + 1] + 1e-16))
  conv = (jnp.concatenate(parts, axis=1) if heads > 1 else parts[0]) + b_r[...]
  conv_o[...] = conv

  @pl.when(i == 0)
  def _():
    ssum_o[...] = jnp.zeros_like(ssum_o[...])
    ssq_o[...] = jnp.zeros_like(ssq_o[...])

  ssum_o[...] += jnp.sum(conv, axis=0, keepdims=True)
  ssq_o[...] += jnp.sum(conv * conv, axis=0, keepdims=True)


def _post_call(heads, num, den, a_s, a_d, m, hw, bias):
  n = num.shape[0]
  nb = n // RB
  full = lambda shp: pl.BlockSpec(shp, lambda i: (0, 0))
  return pl.pallas_call(
      functools.partial(_post_kernel, heads, n),
      grid=(nb,),
      in_specs=[
          pl.BlockSpec((RB, 64), lambda i: (i, 0)),
          pl.BlockSpec((RB, heads), lambda i: (i, 0)),
          pl.BlockSpec((RB, heads), lambda i: (i, 0)),
          pl.BlockSpec((RB, heads), lambda i: (i, 0)),
          full((1, heads)),
          pl.BlockSpec((RB, 64), lambda i: (i, 0)),
          full((1, 64)),
      ],
      out_specs=[
          pl.BlockSpec((RB, 64), lambda i: (i, 0)),
          full((1, 64)), full((1, 64)),
      ],
      out_shape=[
          jax.ShapeDtypeStruct((n, 64), jnp.float32),
          jax.ShapeDtypeStruct((1, 64), jnp.float32),
          jax.ShapeDtypeStruct((1, 64), jnp.float32),
      ],
  )(num, den, a_s, a_d, m, hw, bias)


def _bnprep_kernel(n, conv_r, ssum_r, ssq_r, g_r, be_r, w_r, a_r, b_r,
                   hw_o, as_o, ad_o, m_o):
  i = pl.program_id(0)
  mu = ssum_r[...] / n
  var = ssq_r[...] / n - mu * mu
  y = (conv_r[...] - mu) / jnp.sqrt(var + 1e-5) * g_r[...] + be_r[...]
  h = _elu(y)
  hw = h @ w_r[...]
  hw_o[...] = hw
  asb = hw @ a_r[...]
  adb = hw @ b_r[...]
  as_o[...] = asb
  ad_o[...] = adb

  @pl.when(i == 0)
  def _():
    m_o[...] = jnp.full_like(m_o[...], NEG_INF)

  m_o[...] = jnp.maximum(m_o[...], jnp.max(asb, axis=0, keepdims=True))


def _bnprep_call(conv, ssum, ssq, gamma, beta, w, a, b, heads):
  n = conv.shape[0]
  nb = n // RB
  full = lambda shp: pl.BlockSpec(shp, lambda i: (0, 0))
  return pl.pallas_call(
      functools.partial(_bnprep_kernel, n),
      grid=(nb,),
      in_specs=[
          pl.BlockSpec((RB, 64), lambda i: (i, 0)),
          full((1, 64)), full((1, 64)), full((1, 64)), full((1, 64)),
          full((64, 64)), full((64, heads)), full((64, heads)),
      ],
      out_specs=[
          pl.BlockSpec((RB, 64), lambda i: (i, 0)),
          pl.BlockSpec((RB, heads), lambda i: (i, 0)),
          pl.BlockSpec((RB, heads), lambda i: (i, 0)),
          pl.BlockSpec((1, heads), lambda i: (0, 0)),
      ],
      out_shape=[
          jax.ShapeDtypeStruct((n, 64), jnp.float32),
          jax.ShapeDtypeStruct((n, heads), jnp.float32),
          jax.ShapeDtypeStruct((n, heads), jnp.float32),
          jax.ShapeDtypeStruct((1, heads), jnp.float32),
      ],
  )(conv, ssum, ssq, gamma, beta, w, a, b)


def _pool_kernel(n, g_groups, conv_r, ssum_r, ssq_r, g_r, be_r, bt_r,
                 pmax_o, psum_o, pcnt_o):
  i = pl.program_id(0)
  mu = ssum_r[...] / n
  var = ssq_r[...] / n - mu * mu
  y = (conv_r[...] - mu) / jnp.sqrt(var + 1e-5) * g_r[...] + be_r[...]
  h = _elu(y)
  bt = bt_r[...]

  @pl.when(i == 0)
  def _():
    pmax_o[...] = jnp.full_like(pmax_o[...], NEG_INF)
    psum_o[...] = jnp.zeros_like(psum_o[...])
    pcnt_o[...] = jnp.zeros_like(pcnt_o[...])

  g0 = bt[0, 0]
  g1 = bt[RB - 1, 0]

  def body(g, _):
    mask = bt == g
    hm = jnp.where(mask, h, NEG_INF)
    gmax = jnp.max(hm, axis=0, keepdims=True)
    hs = jnp.where(mask, h, 0.0)
    gsum = jnp.sum(hs, axis=0, keepdims=True)
    gcnt = jnp.sum(jnp.where(mask, 1.0, 0.0))
    pmax_o[pl.ds(g, 1), :] = jnp.maximum(pmax_o[pl.ds(g, 1), :], gmax)
    psum_o[pl.ds(g, 1), :] = psum_o[pl.ds(g, 1), :] + gsum
    pcnt_o[pl.ds(g, 1), :] = pcnt_o[pl.ds(g, 1), :] + gcnt
    return 0

  lax.fori_loop(g0, g1 + 1, body, 0)


def _pool_call(conv, ssum, ssq, gamma, beta, batch2d, g_groups):
  n = conv.shape[0]
  nb = n // RB
  full = lambda shp: pl.BlockSpec(shp, lambda i: (0, 0))
  return pl.pallas_call(
      functools.partial(_pool_kernel, n, g_groups),
      grid=(nb,),
      in_specs=[
          pl.BlockSpec((RB, 64), lambda i: (i, 0)),
          full((1, 64)), full((1, 64)), full((1, 64)), full((1, 64)),
          pl.BlockSpec((RB, 1), lambda i: (i, 0)),
      ],
      out_specs=[
          full((g_groups, 64)), full((g_groups, 64)), full((g_groups, 64)),
      ],
      out_shape=[
          jax.ShapeDtypeStruct((g_groups, 64), jnp.float32),
          jax.ShapeDtypeStruct((g_groups, 64), jnp.float32),
          jax.ShapeDtypeStruct((g_groups, 64), jnp.float32),
      ],
  )(conv, ssum, ssq, gamma, beta, batch2d)


def _final_kernel(pmax_r, psum_r, pcnt_r, w_r, b_r, out_o):
  pmax = pmax_r[...]
  xmax = jnp.where(pmax == NEG_INF, 0.0, pmax)
  xmean = psum_r[...] / jnp.maximum(pcnt_r[...], 1.0)
  comb = jnp.concatenate([xmax, xmean], axis=1)
  out_o[...] = comb @ w_r[...] + b_r[...]


def _final_call(pmax, psum, pcnt, w_out, b_out):
  g = pmax.shape[0]
  return pl.pallas_call(
      _final_kernel,
      out_shape=jax.ShapeDtypeStruct((g, 128), jnp.float32),
  )(pmax, psum, pcnt, w_out, b_out)


def _sc_edge_call(hsrc, srcp, dst3, adp, mrow, *, n, e_real, head_is_core):
  ep = srcp.shape[0]
  et = ep // 16
  n_chunks = et // CE
  drn = 200
  nch = n // drn
  ndr = (nch + 15) // 16
  nden = ((n // 16 + 127) // 128) * 128
  dpt = nden // 16
  mesh = plsc.VectorSubcoreMesh(core_axis_name="c", subcore_axis_name="s")

  @functools.partial(
      pl.kernel,
      mesh=mesh,
      compiler_params=pltpu.CompilerParams(
          needs_layout_passes=False, use_tc_tiling_on_sc=False),
      out_type=[
          jax.ShapeDtypeStruct((2, n, 32), jnp.float32),
          jax.ShapeDtypeStruct((2, nden, 16), jnp.float32),
      ],
      scratch_types=[
          pltpu.VMEM((CE,), jnp.int32),
          pltpu.VMEM((8, 128), jnp.int32),
          pltpu.VMEM((8, 128), jnp.int32),
          pltpu.VMEM((SUB + 16,), jnp.float32),
          pltpu.VMEM((SUB, 48), jnp.float32),
          pltpu.VMEM((SUB, 16), jnp.float32),
          pltpu.VMEM((SUB, 32), jnp.float32),
          pltpu.VMEM((SUB, 16), jnp.float32),
          pltpu.VMEM((1, 128), jnp.int32),
          pltpu.VMEM((drn, 32), jnp.float32),
          pltpu.VMEM((dpt, 16), jnp.float32),
          pltpu.VMEM((16,), jnp.float32),
          pltpu.SemaphoreType.DMA,
          pltpu.VMEM_SHARED((n, 32), jnp.float32),
          pltpu.VMEM_SHARED((nden, 16), jnp.float32),
      ],
  )
  def sc_k(hsrc_hbm, srcp_hbm, dst3_hbm, adp_hbm, m_hbm,
           num_hbm, den_hbm,
           src_v, dst_v, dstg_v, ex_v, rows_v, adrow_v, srows_v, oh_v,
           ddiv_v, bounce_v, dbounce_v, m_v, sem, out_sp, den_sp):
    cid = lax.axis_index("c")
    sid = lax.axis_index("s")
    head = cid if head_is_core else 0
    pltpu.sync_copy(m_hbm.at[head], m_v)
    mvec = m_v[...]
    zvec = jnp.zeros((16,), jnp.float32)
    iota16 = lax.iota(jnp.int32, 16)
    c32 = jnp.full((16,), 32, jnp.int32)
    c0 = jnp.zeros((16,), jnp.int32)

    def zb(r, _):
      bounce_v[r, pl.ds(0, 16)] = zvec
      bounce_v[r, pl.ds(16, 16)] = zvec
      return 0

    lax.fori_loop(0, drn, zb, 0)

    def zd(r, _):
      dbounce_v[r, :] = zvec
      return 0

    lax.fori_loop(0, dpt, zd, 0)

    def zcp(k, _):
      c = sid + 16 * k

      @pl.when(c < nch)
      def _():
        pltpu.sync_copy(bounce_v, out_sp.at[pl.ds(c * drn, drn)])

      return 0

    lax.fori_loop(0, ndr, zcp, 0)
    pltpu.sync_copy(dbounce_v, den_sp.at[pl.ds(sid * dpt, dpt)])
    plsc.subcore_barrier()

    coff = cid * n
    goff = head * n

    def chunk_body(ch, _):
      base = sid * et + ch * CE
      pltpu.sync_copy(srcp_hbm.at[pl.ds(base, CE)], src_v)
      pltpu.sync_copy(dst3_hbm.at[sid, pl.ds(ch * 8, 8)], dst_v)

      def rb(q, _):
        r = q // 8
        lq = q % 8
        src_v[pl.ds(q * 16, 16)] = src_v[pl.ds(q * 16, 16)] + coff
        dstg_v[r, pl.ds(lq * 16, 16)] = (
            dst_v[r, pl.ds(lq * 16, 16)] + goff)
        return 0

      lax.fori_loop(0, CE // 16, rb, 0)

      def sub_body(j, _):
        pltpu.async_copy(
            hsrc_hbm.at[src_v.at[pl.ds(j * SUB, SUB)]], rows_v, sem).wait()
        pltpu.async_copy(
            adp_hbm.at[dstg_v.at[j]], adrow_v, sem).wait()

        def zoh(r, _):
          oh_v[r, :] = zvec
          return 0

        lax.fori_loop(0, SUB, zoh, 0)

        def ohb(l, _):
          rid = iota16 + l * 16
          asg = plsc.load_gather(rows_v, [rid, c32])
          adg = plsc.load_gather(adrow_v, [rid, c0])
          t0 = asg + adg
          al = jnp.maximum(t0, 0.2 * t0)
          t1 = mvec + adg
          sh = jnp.maximum(t1, 0.2 * t1)
          exv = jnp.exp(al - sh)
          gid = iota16 + (base + j * SUB + l * 16)
          exv = jnp.where(gid < e_real, exv, 0.0)
          ex_v[pl.ds(l * 16, 16)] = exv
          d16 = dst_v[j, pl.ds(l * 16, 16)]
          dmod = jnp.bitwise_and(d16, 15)
          ddiv = jnp.right_shift(d16, 4)
          ddiv_v[0, pl.ds(l * 16, 16)] = ddiv
          plsc.store_scatter(oh_v, [rid, dmod], exv)
          return 0

        lax.fori_loop(0, SUB // 16, ohb, 0)

        def scale(e2, _):
          exs = ex_v[pl.ds(e2, 16)][0]
          srows_v[e2, pl.ds(0, 16)] = rows_v[e2, pl.ds(0, 16)] * exs
          srows_v[e2, pl.ds(16, 16)] = rows_v[e2, pl.ds(16, 16)] * exs
          return 0

        lax.fori_loop(0, SUB, scale, 0)

        pltpu.sync_copy(srows_v, out_sp.at[dst_v.at[j]], add=True)
        pltpu.sync_copy(oh_v, den_sp.at[ddiv_v.at[0]], add=True)
        return 0

      lax.fori_loop(0, CE // SUB, sub_body, 0)
      return 0

    lax.fori_loop(0, n_chunks, chunk_body, 0)
    plsc.subcore_barrier()

    def drain(k, _):
      c = sid + 16 * k

      @pl.when(c < nch)
      def _():
        pltpu.sync_copy(out_sp.at[pl.ds(c * drn, drn)], bounce_v)
        pltpu.sync_copy(bounce_v, num_hbm.at[cid, pl.ds(c * drn, drn)])

      return 0

    lax.fori_loop(0, ndr, drain, 0)
    pltpu.sync_copy(den_sp.at[pl.ds(sid * dpt, dpt)], dbounce_v)
    pltpu.sync_copy(dbounce_v, den_hbm.at[cid, pl.ds(sid * dpt, dpt)])

  return sc_k(hsrc, srcp, dst3, adp, mrow)


def kernel(x, edge_index, batch, W_emb, b_emb, W1, att_src1, att_dst1, b1,
           W2, att_src2, att_dst2, b2, gamma1, beta1, gamma2, beta2,
           W_out, b_out):
  n = x.shape[0]
  e = edge_index.shape[1]
  g_groups = 64
  f32 = jnp.float32

  ep = ((e + 16 * CE - 1) // (16 * CE)) * (16 * CE)
  src = edge_index[0]
  dst = edge_index[1]
  srcp = jnp.concatenate([src, jnp.zeros((ep - e,), jnp.int32)])
  dstp = jnp.concatenate([dst, jnp.zeros((ep - e,), jnp.int32)])
  dst3 = dstp.reshape(16, (ep // 16) // 128, 128)
  batch2d = batch.reshape(n, 1)

  a1 = jnp.zeros((64, 2), f32)
  a1 = a1.at[0:32, 0].set(att_src1[0]).at[32:64, 1].set(att_src1[1])
  b1a = jnp.zeros((64, 2), f32)
  b1a = b1a.at[0:32, 0].set(att_dst1[0]).at[32:64, 1].set(att_dst1[1])
  a2 = att_src2.T
  b2a = att_dst2.T

  bemb2d = b_emb.reshape(1, 64)
  b1_2d = b1.reshape(1, 64)
  b2_2d = b2.reshape(1, 64)
  g1_2d = gamma1.reshape(1, 64)
  be1_2d = beta1.reshape(1, 64)
  g2_2d = gamma2.reshape(1, 64)
  be2_2d = beta2.reshape(1, 64)
  bout2d = b_out.reshape(1, 128)

  zpad15 = jnp.zeros((n, 15), f32)

  hw1, as1, ad1, m1 = _prep_call(True, x, W_emb, bemb2d, W1, a1, b1a, 2)
  hsrc1 = jnp.concatenate([
      jnp.concatenate([hw1[:, :32], as1[:, 0:1], zpad15], axis=1),
      jnp.concatenate([hw1[:, 32:], as1[:, 1:2], zpad15], axis=1),
  ], axis=0)
  adp1 = jnp.concatenate([
      jnp.concatenate([ad1[:, 0:1], zpad15], axis=1),
      jnp.concatenate([ad1[:, 1:2], zpad15], axis=1),
  ], axis=0)
  mp1 = jnp.broadcast_to(m1.T, (2, 16))
  num1, den1 = _sc_edge_call(hsrc1, srcp, dst3, adp1, mp1,
                             n=n, e_real=e, head_is_core=True)
  num1f = jnp.concatenate([num1[0], num1[1]], axis=1)
  den1f = jnp.stack(
      [den1[0].reshape(-1)[:n], den1[1].reshape(-1)[:n]], axis=1)
  conv1, s1, q1 = _post_call(2, num1f, den1f, as1, ad1, m1, hw1, b1_2d)

  hw2, as2, ad2, m2 = _bnprep_call(conv1, s1, q1, g1_2d, be1_2d, W2, a2,
                                   b2a, 1)
  hsrc2 = jnp.concatenate([
      jnp.concatenate([hw2[:, :32], as2, zpad15], axis=1),
      jnp.concatenate([hw2[:, 32:], as2, zpad15], axis=1),
  ], axis=0)
  adp2 = jnp.concatenate([ad2, zpad15], axis=1)
  mp2 = jnp.broadcast_to(m2.T, (1, 16))
  num2, den2 = _sc_edge_call(hsrc2, srcp, dst3, adp2, mp2,
                             n=n, e_real=e, head_is_core=False)
  num2f = jnp.concatenate([num2[0], num2[1]], axis=1)
  den2f = den2[0].reshape(-1)[:n].reshape(n, 1)
  conv2, s2, q2 = _post_call(1, num2f, den2f, as2, ad2, m2, hw2, b2_2d)

  pmax, psum, pcnt = _pool_call(conv2, s2, q2, g2_2d, be2_2d, batch2d,
                                g_groups)
  return _final_call(pmax, psum, pcnt, W_out, b_out.reshape(1, 128))


# speedup vs baseline: 46.0826x; 1.5787x over previous
import functools

import jax
import jax.numpy as jnp
from jax import lax
from jax.experimental import pallas as pl
from jax.experimental.pallas import tpu as pltpu
from jax.experimental.pallas import tpu_sc as plsc

RB = 1000
CE = 1024
SUB = 128
NEG_INF = float("-inf")


def _elu(x):
  return jnp.where(x > 0, x, jnp.exp(jnp.minimum(x, 0.0)) - 1.0)


def _prep_kernel(embed, h_r, wemb_r, bemb_r, w_r, a_r, b_r,
                 hw_o, as_o, ad_o, m_o):
  i = pl.program_id(0)
  h = h_r[...]
  if embed:
    h = _elu(h @ wemb_r[...] + bemb_r[...])
  hw = h @ w_r[...]
  hw_o[...] = hw
  asb = hw @ a_r[...]
  adb = hw @ b_r[...]
  as_o[...] = asb
  ad_o[...] = adb

  @pl.when(i == 0)
  def _():
    m_o[...] = jnp.full_like(m_o[...], NEG_INF)

  m_o[...] = jnp.maximum(m_o[...], jnp.max(asb, axis=0, keepdims=True))


def _prep_call(embed, h_in, wemb, bemb, w, a, b, heads):
  n = h_in.shape[0]
  nb = n // RB
  cin = h_in.shape[1]
  full = lambda shp: pl.BlockSpec(shp, lambda i: (0, 0))
  return pl.pallas_call(
      functools.partial(_prep_kernel, embed),
      grid=(nb,),
      in_specs=[
          pl.BlockSpec((RB, cin), lambda i: (i, 0)),
          full(wemb.shape), full(bemb.shape), full(w.shape),
          full(a.shape), full(b.shape),
      ],
      out_specs=[
          pl.BlockSpec((RB, 64), lambda i: (i, 0)),
          pl.BlockSpec((RB, heads), lambda i: (i, 0)),
          pl.BlockSpec((RB, heads), lambda i: (i, 0)),
          pl.BlockSpec((1, heads), lambda i: (0, 0)),
      ],
      out_shape=[
          jax.ShapeDtypeStruct((n, 64), jnp.float32),
          jax.ShapeDtypeStruct((n, heads), jnp.float32),
          jax.ShapeDtypeStruct((n, heads), jnp.float32),
          jax.ShapeDtypeStruct((1, heads), jnp.float32),
      ],
  )(h_in, wemb, bemb, w, a, b)


def _post_kernel(heads, n, num_r, den_r, as_r, ad_r, m_r, hw_r, b_r,
                 conv_o, ssum_o, ssq_o):
  i = pl.program_id(0)
  c = 64 // heads
  asb = as_r[...]
  adb = ad_r[...]
  m = m_r[...]
  t = asb + adb
  al = jnp.maximum(t, 0.2 * t)
  t2 = m + adb
  d2 = jnp.maximum(t2, 0.2 * t2)
  sex = jnp.exp(al - d2)
  den = den_r[...] + sex
  hw = hw_r[...]
  num = num_r[...]
  parts = []
  for h in range(heads):
    nh = num[:, h * c:(h + 1) * c] + sex[:, h:h + 1] * hw[:, h * c:(h + 1) * c]
    parts.append(nh / (den[:, h:h + 1] + 1e-16))
  conv = (jnp.concatenate(parts, axis=1) if heads > 1 else parts[0]) + b_r[...]
  conv_o[...] = conv

  @pl.when(i == 0)
  def _():
    ssum_o[...] = jnp.zeros_like(ssum_o[...])
    ssq_o[...] = jnp.zeros_like(ssq_o[...])

  ssum_o[...] += jnp.sum(conv, axis=0, keepdims=True)
  ssq_o[...] += jnp.sum(conv * conv, axis=0, keepdims=True)


def _post_call(heads, num, den, a_s, a_d, m, hw, bias):
  n = num.shape[0]
  nb = n // RB
  full = lambda shp: pl.BlockSpec(shp, lambda i: (0, 0))
  return pl.pallas_call(
      functools.partial(_post_kernel, heads, n),
      grid=(nb,),
      in_specs=[
          pl.BlockSpec((RB, 64), lambda i: (i, 0)),
          pl.BlockSpec((RB, heads), lambda i: (i, 0)),
          pl.BlockSpec((RB, heads), lambda i: (i, 0)),
          pl.BlockSpec((RB, heads), lambda i: (i, 0)),
          full((1, heads)),
          pl.BlockSpec((RB, 64), lambda i: (i, 0)),
          full((1, 64)),
      ],
      out_specs=[
          pl.BlockSpec((RB, 64), lambda i: (i, 0)),
          full((1, 64)), full((1, 64)),
      ],
      out_shape=[
          jax.ShapeDtypeStruct((n, 64), jnp.float32),
          jax.ShapeDtypeStruct((1, 64), jnp.float32),
          jax.ShapeDtypeStruct((1, 64), jnp.float32),
      ],
  )(num, den, a_s, a_d, m, hw, bias)


def _bnprep_kernel(n, conv_r, ssum_r, ssq_r, g_r, be_r, w_r, a_r, b_r,
                   hw_o, as_o, ad_o, m_o):
  i = pl.program_id(0)
  mu = ssum_r[...] / n
  var = ssq_r[...] / n - mu * mu
  y = (conv_r[...] - mu) / jnp.sqrt(var + 1e-5) * g_r[...] + be_r[...]
  h = _elu(y)
  hw = h @ w_r[...]
  hw_o[...] = hw
  asb = hw @ a_r[...]
  adb = hw @ b_r[...]
  as_o[...] = asb
  ad_o[...] = adb

  @pl.when(i == 0)
  def _():
    m_o[...] = jnp.full_like(m_o[...], NEG_INF)

  m_o[...] = jnp.maximum(m_o[...], jnp.max(asb, axis=0, keepdims=True))


def _bnprep_call(conv, ssum, ssq, gamma, beta, w, a, b, heads):
  n = conv.shape[0]
  nb = n // RB
  full = lambda shp: pl.BlockSpec(shp, lambda i: (0, 0))
  return pl.pallas_call(
      functools.partial(_bnprep_kernel, n),
      grid=(nb,),
      in_specs=[
          pl.BlockSpec((RB, 64), lambda i: (i, 0)),
          full((1, 64)), full((1, 64)), full((1, 64)), full((1, 64)),
          full((64, 64)), full((64, heads)), full((64, heads)),
      ],
      out_specs=[
          pl.BlockSpec((RB, 64), lambda i: (i, 0)),
          pl.BlockSpec((RB, heads), lambda i: (i, 0)),
          pl.BlockSpec((RB, heads), lambda i: (i, 0)),
          pl.BlockSpec((1, heads), lambda i: (0, 0)),
      ],
      out_shape=[
          jax.ShapeDtypeStruct((n, 64), jnp.float32),
          jax.ShapeDtypeStruct((n, heads), jnp.float32),
          jax.ShapeDtypeStruct((n, heads), jnp.float32),
          jax.ShapeDtypeStruct((1, heads), jnp.float32),
      ],
  )(conv, ssum, ssq, gamma, beta, w, a, b)


def _pool_kernel(n, g_groups, conv_r, ssum_r, ssq_r, g_r, be_r, bt_r,
                 pmax_o, psum_o, pcnt_o):
  i = pl.program_id(0)
  mu = ssum_r[...] / n
  var = ssq_r[...] / n - mu * mu
  y = (conv_r[...] - mu) / jnp.sqrt(var + 1e-5) * g_r[...] + be_r[...]
  h = _elu(y)
  bt = bt_r[...]

  @pl.when(i == 0)
  def _():
    pmax_o[...] = jnp.full_like(pmax_o[...], NEG_INF)
    psum_o[...] = jnp.zeros_like(psum_o[...])
    pcnt_o[...] = jnp.zeros_like(pcnt_o[...])

  g0 = bt[0, 0]
  g1 = bt[RB - 1, 0]

  def body(g, _):
    mask = bt == g
    hm = jnp.where(mask, h, NEG_INF)
    gmax = jnp.max(hm, axis=0, keepdims=True)
    hs = jnp.where(mask, h, 0.0)
    gsum = jnp.sum(hs, axis=0, keepdims=True)
    gcnt = jnp.sum(jnp.where(mask, 1.0, 0.0))
    pmax_o[pl.ds(g, 1), :] = jnp.maximum(pmax_o[pl.ds(g, 1), :], gmax)
    psum_o[pl.ds(g, 1), :] = psum_o[pl.ds(g, 1), :] + gsum
    pcnt_o[pl.ds(g, 1), :] = pcnt_o[pl.ds(g, 1), :] + gcnt
    return 0

  lax.fori_loop(g0, g1 + 1, body, 0)


def _pool_call(conv, ssum, ssq, gamma, beta, batch2d, g_groups):
  n = conv.shape[0]
  nb = n // RB
  full = lambda shp: pl.BlockSpec(shp, lambda i: (0, 0))
  return pl.pallas_call(
      functools.partial(_pool_kernel, n, g_groups),
      grid=(nb,),
      in_specs=[
          pl.BlockSpec((RB, 64), lambda i: (i, 0)),
          full((1, 64)), full((1, 64)), full((1, 64)), full((1, 64)),
          pl.BlockSpec((RB, 1), lambda i: (i, 0)),
      ],
      out_specs=[
          full((g_groups, 64)), full((g_groups, 64)), full((g_groups, 64)),
      ],
      out_shape=[
          jax.ShapeDtypeStruct((g_groups, 64), jnp.float32),
          jax.ShapeDtypeStruct((g_groups, 64), jnp.float32),
          jax.ShapeDtypeStruct((g_groups, 64), jnp.float32),
      ],
  )(conv, ssum, ssq, gamma, beta, batch2d)


def _final_kernel(pmax_r, psum_r, pcnt_r, w_r, b_r, out_o):
  pmax = pmax_r[...]
  xmax = jnp.where(pmax == NEG_INF, 0.0, pmax)
  xmean = psum_r[...] / jnp.maximum(pcnt_r[...], 1.0)
  comb = jnp.concatenate([xmax, xmean], axis=1)
  out_o[...] = comb @ w_r[...] + b_r[...]


def _final_call(pmax, psum, pcnt, w_out, b_out):
  g = pmax.shape[0]
  return pl.pallas_call(
      _final_kernel,
      out_shape=jax.ShapeDtypeStruct((g, 128), jnp.float32),
  )(pmax, psum, pcnt, w_out, b_out)


def _sc_edge_call(hsrc, srcp, dst3, adp, mrow, *, n, e_real, head_is_core):
  ep = srcp.shape[0]
  et = ep // 16
  n_chunks = et // CE
  nsub = CE // SUB
  drn = 80
  nch = n // drn
  ndr = (nch + 15) // 16
  nden = ((n // 16 + 127) // 128) * 128
  dch = nden // SUB
  ndd = (dch + 15) // 16
  mesh = plsc.VectorSubcoreMesh(core_axis_name="c", subcore_axis_name="s")

  @functools.partial(
      pl.kernel,
      mesh=mesh,
      compiler_params=pltpu.CompilerParams(
          needs_layout_passes=False, use_tc_tiling_on_sc=False),
      out_type=[
          jax.ShapeDtypeStruct((2, n, 32), jnp.float32),
          jax.ShapeDtypeStruct((2, nden, 16), jnp.float32),
      ],
      scratch_types=[
          pltpu.VMEM((CE,), jnp.int32),
          pltpu.VMEM((CE // 128, 128), jnp.int32),
          pltpu.VMEM((CE // 128, 128), jnp.int32),
          pltpu.VMEM((SUB + 16,), jnp.float32),
          pltpu.VMEM((SUB, 48), jnp.float32),
          pltpu.VMEM((SUB, 48), jnp.float32),
          pltpu.VMEM((SUB, 16), jnp.float32),
          pltpu.VMEM((SUB, 16), jnp.float32),
          pltpu.VMEM((SUB, 32), jnp.float32),
          pltpu.VMEM((SUB, 16), jnp.float32),
          pltpu.VMEM((1, 128), jnp.int32),
          pltpu.VMEM((16,), jnp.float32),
          pltpu.SemaphoreType.DMA,
          pltpu.SemaphoreType.DMA,
          pltpu.SemaphoreType.DMA,
          pltpu.SemaphoreType.DMA,
          pltpu.VMEM_SHARED((n, 32), jnp.float32),
          pltpu.VMEM_SHARED((nden, 16), jnp.float32),
      ],
  )
  def sc_k(hsrc_hbm, srcp_hbm, dst3_hbm, adp_hbm, m_hbm,
           num_hbm, den_hbm,
           src_v, dst_v, dstg_v, ex_v, rows_a, rows_b, adrow_a, adrow_b,
           srows_v, oh_v, ddiv_v, m_v, sem_ra, sem_rb, sem_aa, sem_ab,
           out_sp, den_sp):
    cid = lax.axis_index("c")
    sid = lax.axis_index("s")
    head = cid if head_is_core else 0
    pltpu.sync_copy(m_hbm.at[head], m_v)
    mvec = m_v[...]
    zvec = jnp.zeros((16,), jnp.float32)
    iota16 = lax.iota(jnp.int32, 16)
    c32 = jnp.full((16,), 32, jnp.int32)
    c0 = jnp.zeros((16,), jnp.int32)
    slots = ((rows_a, adrow_a, sem_ra, sem_aa),
             (rows_b, adrow_b, sem_rb, sem_ab))

    def zs(r, _):
      srows_v[r, pl.ds(0, 16)] = zvec
      srows_v[r, pl.ds(16, 16)] = zvec
      oh_v[r, :] = zvec
      return 0

    lax.fori_loop(0, SUB, zs, 0)

    def zcp(k, _):
      c = sid + 16 * k

      @pl.when(c < nch)
      def _():
        pltpu.sync_copy(srows_v.at[pl.ds(0, drn)],
                        out_sp.at[pl.ds(c * drn, drn)])

      return 0

    lax.fori_loop(0, ndr, zcp, 0)

    def zcd(k, _):
      c = sid + 16 * k

      @pl.when(c < dch)
      def _():
        pltpu.sync_copy(oh_v, den_sp.at[pl.ds(c * SUB, SUB)])

      return 0

    lax.fori_loop(0, ndd, zcd, 0)
    plsc.subcore_barrier()

    coff = cid * n
    goff = head * n

    def chunk_body(ch, _):
      base = sid * et + ch * CE
      pltpu.sync_copy(srcp_hbm.at[pl.ds(base, CE)], src_v)
      pltpu.sync_copy(dst3_hbm.at[sid, pl.ds(ch * (CE // 128), CE // 128)],
                      dst_v)

      def rb(q, _):
        r = q // 8
        lq = q % 8
        src_v[pl.ds(q * 16, 16)] = src_v[pl.ds(q * 16, 16)] + coff
        dstg_v[r, pl.ds(lq * 16, 16)] = (
            dst_v[r, pl.ds(lq * 16, 16)] + goff)
        return 0

      lax.fori_loop(0, CE // 16, rb, 0)

      def issue(j, slot):
        rows_s, adrow_s, sem_r, sem_a = slot
        h1 = pltpu.async_copy(
            hsrc_hbm.at[src_v.at[pl.ds(j * SUB, SUB)]], rows_s, sem_r)
        h2 = pltpu.async_copy(adp_hbm.at[dstg_v.at[j]], adrow_s, sem_a)
        return (h1, h2)

      pend = issue(0, slots[0])
      for j in range(nsub):
        if j + 1 < nsub:
          nxt = issue(j + 1, slots[(j + 1) % 2])
        rows_s, adrow_s, _, _ = slots[j % 2]
        pend[0].wait()
        pend[1].wait()

        def ohb(l, _):
          rid = iota16 + l * 16
          asg = plsc.load_gather(rows_s, [rid, c32])
          adg = plsc.load_gather(adrow_s, [rid, c0])
          t0 = asg + adg
          al = jnp.maximum(t0, 0.2 * t0)
          t1 = mvec + adg
          sh = jnp.maximum(t1, 0.2 * t1)
          exv = jnp.exp(al - sh)
          gid = iota16 + (base + j * SUB + l * 16)
          exv = jnp.where(gid < e_real, exv, 0.0)
          ex_v[pl.ds(l * 16, 16)] = exv
          d16 = dst_v[j, pl.ds(l * 16, 16)]
          dmod = jnp.bitwise_and(d16, 15)
          ddiv = jnp.right_shift(d16, 4)
          ddiv_v[0, pl.ds(l * 16, 16)] = ddiv
          plsc.store_scatter(oh_v, [rid, dmod], exv)
          return 0

        lax.fori_loop(0, SUB // 16, ohb, 0)

        def scale(e2, _):
          exs = ex_v[pl.ds(e2, 16)][0]
          srows_v[e2, pl.ds(0, 16)] = rows_s[e2, pl.ds(0, 16)] * exs
          srows_v[e2, pl.ds(16, 16)] = rows_s[e2, pl.ds(16, 16)] * exs
          return 0

        lax.fori_loop(0, SUB, scale, 0, unroll=8)

        pltpu.sync_copy(srows_v, out_sp.at[dst_v.at[j]], add=True)
        pltpu.sync_copy(oh_v, den_sp.at[ddiv_v.at[0]], add=True)

        def rz(l, _):
          rid = iota16 + l * 16
          d16 = dst_v[j, pl.ds(l * 16, 16)]
          dmod = jnp.bitwise_and(d16, 15)
          plsc.store_scatter(oh_v, [rid, dmod], zvec)
          return 0

        lax.fori_loop(0, SUB // 16, rz, 0)
        if j + 1 < nsub:
          pend = nxt
      return 0

    lax.fori_loop(0, n_chunks, chunk_body, 0)
    plsc.subcore_barrier()

    def drain(k, _):
      c = sid + 16 * k

      @pl.when(c < nch)
      def _():
        pltpu.sync_copy(out_sp.at[pl.ds(c * drn, drn)],
                        srows_v.at[pl.ds(0, drn)])
        pltpu.sync_copy(srows_v.at[pl.ds(0, drn)],
                        num_hbm.at[cid, pl.ds(c * drn, drn)])

      return 0

    lax.fori_loop(0, ndr, drain, 0)

    def draind(k, _):
      c = sid + 16 * k

      @pl.when(c < dch)
      def _():
        pltpu.sync_copy(den_sp.at[pl.ds(c * SUB, SUB)], oh_v)
        pltpu.sync_copy(oh_v, den_hbm.at[cid, pl.ds(c * SUB, SUB)])

      return 0

    lax.fori_loop(0, ndd, draind, 0)

  return sc_k(hsrc, srcp, dst3, adp, mrow)


def kernel(x, edge_index, batch, W_emb, b_emb, W1, att_src1, att_dst1, b1,
           W2, att_src2, att_dst2, b2, gamma1, beta1, gamma2, beta2,
           W_out, b_out):
  n = x.shape[0]
  e = edge_index.shape[1]
  g_groups = 64
  f32 = jnp.float32

  ep = ((e + 16 * CE - 1) // (16 * CE)) * (16 * CE)
  src = edge_index[0]
  dst = edge_index[1]
  srcp = jnp.concatenate([src, jnp.zeros((ep - e,), jnp.int32)])
  dstp = jnp.concatenate([dst, jnp.zeros((ep - e,), jnp.int32)])
  dst3 = dstp.reshape(16, (ep // 16) // 128, 128)
  batch2d = batch.reshape(n, 1)

  a1 = jnp.zeros((64, 2), f32)
  a1 = a1.at[0:32, 0].set(att_src1[0]).at[32:64, 1].set(att_src1[1])
  b1a = jnp.zeros((64, 2), f32)
  b1a = b1a.at[0:32, 0].set(att_dst1[0]).at[32:64, 1].set(att_dst1[1])
  a2 = att_src2.T
  b2a = att_dst2.T

  bemb2d = b_emb.reshape(1, 64)
  b1_2d = b1.reshape(1, 64)
  b2_2d = b2.reshape(1, 64)
  g1_2d = gamma1.reshape(1, 64)
  be1_2d = beta1.reshape(1, 64)
  g2_2d = gamma2.reshape(1, 64)
  be2_2d = beta2.reshape(1, 64)
  bout2d = b_out.reshape(1, 128)

  zpad15 = jnp.zeros((n, 15), f32)

  hw1, as1, ad1, m1 = _prep_call(True, x, W_emb, bemb2d, W1, a1, b1a, 2)
  hsrc1 = jnp.concatenate([
      jnp.concatenate([hw1[:, :32], as1[:, 0:1], zpad15], axis=1),
      jnp.concatenate([hw1[:, 32:], as1[:, 1:2], zpad15], axis=1),
  ], axis=0)
  adp1 = jnp.concatenate([
      jnp.concatenate([ad1[:, 0:1], zpad15], axis=1),
      jnp.concatenate([ad1[:, 1:2], zpad15], axis=1),
  ], axis=0)
  mp1 = jnp.broadcast_to(m1.T, (2, 16))
  num1, den1 = _sc_edge_call(hsrc1, srcp, dst3, adp1, mp1,
                             n=n, e_real=e, head_is_core=True)
  num1f = jnp.concatenate([num1[0], num1[1]], axis=1)
  den1f = jnp.stack(
      [den1[0].reshape(-1)[:n], den1[1].reshape(-1)[:n]], axis=1)
  conv1, s1, q1 = _post_call(2, num1f, den1f, as1, ad1, m1, hw1, b1_2d)

  hw2, as2, ad2, m2 = _bnprep_call(conv1, s1, q1, g1_2d, be1_2d, W2, a2,
                                   b2a, 1)
  hsrc2 = jnp.concatenate([
      jnp.concatenate([hw2[:, :32], as2, zpad15], axis=1),
      jnp.concatenate([hw2[:, 32:], as2, zpad15], axis=1),
  ], axis=0)
  adp2 = jnp.concatenate([ad2, zpad15], axis=1)
  mp2 = jnp.broadcast_to(m2.T, (1, 16))
  num2, den2 = _sc_edge_call(hsrc2, srcp, dst3, adp2, mp2,
                             n=n, e_real=e, head_is_core=False)
  num2f = jnp.concatenate([num2[0], num2[1]], axis=1)
  den2f = den2[0].reshape(-1)[:n].reshape(n, 1)
  conv2, s2, q2 = _post_call(1, num2f, den2f, as2, ad2, m2, hw2, b2_2d)

  pmax, psum, pcnt = _pool_call(conv2, s2, q2, g2_2d, be2_2d, batch2d,
                                g_groups)
  return _final_call(pmax, psum, pcnt, W_out, b_out.reshape(1, 128))


# speedup vs baseline: 48.2686x; 1.0474x over previous
import functools

import jax
import jax.numpy as jnp
from jax import lax
from jax.experimental import pallas as pl
from jax.experimental.pallas import tpu as pltpu
from jax.experimental.pallas import tpu_sc as plsc

RB = 1000
CE = 1024
SUB = 128
NEG_INF = float("-inf")


def _elu(x):
  return jnp.where(x > 0, x, jnp.exp(jnp.minimum(x, 0.0)) - 1.0)


def _prep_kernel(embed, h_r, wemb_r, bemb_r, w_r, a_r, b_r,
                 hw_o, as_o, ad_o, m_o):
  i = pl.program_id(0)
  h = h_r[...]
  if embed:
    h = _elu(h @ wemb_r[...] + bemb_r[...])
  hw = h @ w_r[...]
  hw_o[...] = hw
  asb = hw @ a_r[...]
  adb = hw @ b_r[...]
  as_o[...] = asb
  ad_o[...] = adb

  @pl.when(i == 0)
  def _():
    m_o[...] = jnp.full_like(m_o[...], NEG_INF)

  m_o[...] = jnp.maximum(m_o[...], jnp.max(asb, axis=0, keepdims=True))


def _prep_call(embed, h_in, wemb, bemb, w, a, b, heads):
  n = h_in.shape[0]
  nb = n // RB
  cin = h_in.shape[1]
  full = lambda shp: pl.BlockSpec(shp, lambda i: (0, 0))
  return pl.pallas_call(
      functools.partial(_prep_kernel, embed),
      grid=(nb,),
      in_specs=[
          pl.BlockSpec((RB, cin), lambda i: (i, 0)),
          full(wemb.shape), full(bemb.shape), full(w.shape),
          full(a.shape), full(b.shape),
      ],
      out_specs=[
          pl.BlockSpec((RB, 64), lambda i: (i, 0)),
          pl.BlockSpec((RB, heads), lambda i: (i, 0)),
          pl.BlockSpec((RB, heads), lambda i: (i, 0)),
          pl.BlockSpec((1, heads), lambda i: (0, 0)),
      ],
      out_shape=[
          jax.ShapeDtypeStruct((n, 64), jnp.float32),
          jax.ShapeDtypeStruct((n, heads), jnp.float32),
          jax.ShapeDtypeStruct((n, heads), jnp.float32),
          jax.ShapeDtypeStruct((1, heads), jnp.float32),
      ],
  )(h_in, wemb, bemb, w, a, b)


def _post_kernel(heads, n, num_r, den_r, as_r, ad_r, m_r, hw_r, b_r,
                 conv_o, ssum_o, ssq_o):
  i = pl.program_id(0)
  c = 64 // heads
  asb = as_r[...]
  adb = ad_r[...]
  m = m_r[...]
  t = asb + adb
  al = jnp.maximum(t, 0.2 * t)
  t2 = m + adb
  d2 = jnp.maximum(t2, 0.2 * t2)
  sex = jnp.exp(al - d2)
  den = den_r[...] + sex
  hw = hw_r[...]
  num = num_r[...]
  parts = []
  for h in range(heads):
    nh = num[:, h * c:(h + 1) * c] + sex[:, h:h + 1] * hw[:, h * c:(h + 1) * c]
    parts.append(nh / (den[:, h:h + 1] + 1e-16))
  conv = (jnp.concatenate(parts, axis=1) if heads > 1 else parts[0]) + b_r[...]
  conv_o[...] = conv

  @pl.when(i == 0)
  def _():
    ssum_o[...] = jnp.zeros_like(ssum_o[...])
    ssq_o[...] = jnp.zeros_like(ssq_o[...])

  ssum_o[...] += jnp.sum(conv, axis=0, keepdims=True)
  ssq_o[...] += jnp.sum(conv * conv, axis=0, keepdims=True)


def _post_call(heads, num, den, a_s, a_d, m, hw, bias):
  n = num.shape[0]
  nb = n // RB
  full = lambda shp: pl.BlockSpec(shp, lambda i: (0, 0))
  return pl.pallas_call(
      functools.partial(_post_kernel, heads, n),
      grid=(nb,),
      in_specs=[
          pl.BlockSpec((RB, 64), lambda i: (i, 0)),
          pl.BlockSpec((RB, heads), lambda i: (i, 0)),
          pl.BlockSpec((RB, heads), lambda i: (i, 0)),
          pl.BlockSpec((RB, heads), lambda i: (i, 0)),
          full((1, heads)),
          pl.BlockSpec((RB, 64), lambda i: (i, 0)),
          full((1, 64)),
      ],
      out_specs=[
          pl.BlockSpec((RB, 64), lambda i: (i, 0)),
          full((1, 64)), full((1, 64)),
      ],
      out_shape=[
          jax.ShapeDtypeStruct((n, 64), jnp.float32),
          jax.ShapeDtypeStruct((1, 64), jnp.float32),
          jax.ShapeDtypeStruct((1, 64), jnp.float32),
      ],
  )(num, den, a_s, a_d, m, hw, bias)


def _bnprep_kernel(n, conv_r, ssum_r, ssq_r, g_r, be_r, w_r, a_r, b_r,
                   hw_o, as_o, ad_o, m_o):
  i = pl.program_id(0)
  mu = ssum_r[...] / n
  var = ssq_r[...] / n - mu * mu
  y = (conv_r[...] - mu) / jnp.sqrt(var + 1e-5) * g_r[...] + be_r[...]
  h = _elu(y)
  hw = h @ w_r[...]
  hw_o[...] = hw
  asb = hw @ a_r[...]
  adb = hw @ b_r[...]
  as_o[...] = asb
  ad_o[...] = adb

  @pl.when(i == 0)
  def _():
    m_o[...] = jnp.full_like(m_o[...], NEG_INF)

  m_o[...] = jnp.maximum(m_o[...], jnp.max(asb, axis=0, keepdims=True))


def _bnprep_call(conv, ssum, ssq, gamma, beta, w, a, b, heads):
  n = conv.shape[0]
  nb = n // RB
  full = lambda shp: pl.BlockSpec(shp, lambda i: (0, 0))
  return pl.pallas_call(
      functools.partial(_bnprep_kernel, n),
      grid=(nb,),
      in_specs=[
          pl.BlockSpec((RB, 64), lambda i: (i, 0)),
          full((1, 64)), full((1, 64)), full((1, 64)), full((1, 64)),
          full((64, 64)), full((64, heads)), full((64, heads)),
      ],
      out_specs=[
          pl.BlockSpec((RB, 64), lambda i: (i, 0)),
          pl.BlockSpec((RB, heads), lambda i: (i, 0)),
          pl.BlockSpec((RB, heads), lambda i: (i, 0)),
          pl.BlockSpec((1, heads), lambda i: (0, 0)),
      ],
      out_shape=[
          jax.ShapeDtypeStruct((n, 64), jnp.float32),
          jax.ShapeDtypeStruct((n, heads), jnp.float32),
          jax.ShapeDtypeStruct((n, heads), jnp.float32),
          jax.ShapeDtypeStruct((1, heads), jnp.float32),
      ],
  )(conv, ssum, ssq, gamma, beta, w, a, b)


def _pool_kernel(n, g_groups, conv_r, ssum_r, ssq_r, g_r, be_r, bt_r,
                 pmax_o, psum_o, pcnt_o):
  i = pl.program_id(0)
  mu = ssum_r[...] / n
  var = ssq_r[...] / n - mu * mu
  y = (conv_r[...] - mu) / jnp.sqrt(var + 1e-5) * g_r[...] + be_r[...]
  h = _elu(y)
  bt = bt_r[...]

  @pl.when(i == 0)
  def _():
    pmax_o[...] = jnp.full_like(pmax_o[...], NEG_INF)
    psum_o[...] = jnp.zeros_like(psum_o[...])
    pcnt_o[...] = jnp.zeros_like(pcnt_o[...])

  g0 = bt[0, 0]
  g1 = bt[RB - 1, 0]

  def body(g, _):
    mask = bt == g
    hm = jnp.where(mask, h, NEG_INF)
    gmax = jnp.max(hm, axis=0, keepdims=True)
    hs = jnp.where(mask, h, 0.0)
    gsum = jnp.sum(hs, axis=0, keepdims=True)
    gcnt = jnp.sum(jnp.where(mask, 1.0, 0.0))
    pmax_o[pl.ds(g, 1), :] = jnp.maximum(pmax_o[pl.ds(g, 1), :], gmax)
    psum_o[pl.ds(g, 1), :] = psum_o[pl.ds(g, 1), :] + gsum
    pcnt_o[pl.ds(g, 1), :] = pcnt_o[pl.ds(g, 1), :] + gcnt
    return 0

  lax.fori_loop(g0, g1 + 1, body, 0)


def _pool_call(conv, ssum, ssq, gamma, beta, batch2d, g_groups):
  n = conv.shape[0]
  nb = n // RB
  full = lambda shp: pl.BlockSpec(shp, lambda i: (0, 0))
  return pl.pallas_call(
      functools.partial(_pool_kernel, n, g_groups),
      grid=(nb,),
      in_specs=[
          pl.BlockSpec((RB, 64), lambda i: (i, 0)),
          full((1, 64)), full((1, 64)), full((1, 64)), full((1, 64)),
          pl.BlockSpec((RB, 1), lambda i: (i, 0)),
      ],
      out_specs=[
          full((g_groups, 64)), full((g_groups, 64)), full((g_groups, 64)),
      ],
      out_shape=[
          jax.ShapeDtypeStruct((g_groups, 64), jnp.float32),
          jax.ShapeDtypeStruct((g_groups, 64), jnp.float32),
          jax.ShapeDtypeStruct((g_groups, 64), jnp.float32),
      ],
  )(conv, ssum, ssq, gamma, beta, batch2d)


def _final_kernel(pmax_r, psum_r, pcnt_r, w_r, b_r, out_o):
  pmax = pmax_r[...]
  xmax = jnp.where(pmax == NEG_INF, 0.0, pmax)
  xmean = psum_r[...] / jnp.maximum(pcnt_r[...], 1.0)
  comb = jnp.concatenate([xmax, xmean], axis=1)
  out_o[...] = comb @ w_r[...] + b_r[...]


def _final_call(pmax, psum, pcnt, w_out, b_out):
  g = pmax.shape[0]
  return pl.pallas_call(
      _final_kernel,
      out_shape=jax.ShapeDtypeStruct((g, 128), jnp.float32),
  )(pmax, psum, pcnt, w_out, b_out)


def _sc_edge_call(hsrc, srcp, dst3, adp, mrow, *, n, e_real, head_is_core):
  ep = srcp.shape[0]
  et = ep // 16
  n_chunks = et // CE
  nsub = CE // SUB
  drn = 80
  nch = n // drn
  ndr = (nch + 15) // 16
  nden = ((n // 16 + 127) // 128) * 128
  dch = nden // SUB
  ndd = (dch + 15) // 16
  mesh = plsc.VectorSubcoreMesh(core_axis_name="c", subcore_axis_name="s")

  @functools.partial(
      pl.kernel,
      mesh=mesh,
      compiler_params=pltpu.CompilerParams(
          needs_layout_passes=False, use_tc_tiling_on_sc=False),
      out_type=[
          jax.ShapeDtypeStruct((2, n, 32), jnp.float32),
          jax.ShapeDtypeStruct((2, nden, 16), jnp.float32),
      ],
      scratch_types=[
          pltpu.VMEM((CE,), jnp.int32),
          pltpu.VMEM((CE // 128, 128), jnp.int32),
          pltpu.VMEM((SUB + 16,), jnp.float32),
          pltpu.VMEM((SUB, 48), jnp.float32),
          pltpu.VMEM((SUB, 48), jnp.float32),
          pltpu.VMEM((SUB, 16), jnp.float32),
          pltpu.VMEM((SUB, 16), jnp.float32),
          pltpu.VMEM((SUB, 32), jnp.float32),
          pltpu.VMEM((SUB, 16), jnp.float32),
          pltpu.VMEM((SUB, 16), jnp.float32),
          pltpu.VMEM((1, 128), jnp.int32),
          pltpu.VMEM((1, 128), jnp.int32),
          pltpu.VMEM((16,), jnp.float32),
          pltpu.SemaphoreType.DMA,
          pltpu.SemaphoreType.DMA,
          pltpu.SemaphoreType.DMA,
          pltpu.SemaphoreType.DMA,
          pltpu.SemaphoreType.DMA,
          pltpu.SemaphoreType.DMA,
          pltpu.VMEM_SHARED((n, 32), jnp.float32),
          pltpu.VMEM_SHARED((nden, 16), jnp.float32),
      ],
  )
  def sc_k(hsrc_hbm, srcp_hbm, dst3_hbm, adp_hbm, m_hbm,
           num_hbm, den_hbm,
           src_v, dst_v, ex_v, rows_a, rows_b, adrow_a, adrow_b,
           srows_v, oh_a, oh_b, ddiv_a, ddiv_b, m_v,
           sem_ra, sem_rb, sem_aa, sem_ab, sem_oa, sem_ob,
           out_sp, den_sp):
    cid = lax.axis_index("c")
    sid = lax.axis_index("s")
    head = cid if head_is_core else 0
    pltpu.sync_copy(m_hbm.at[head], m_v)
    mvec = m_v[...]
    zvec = jnp.zeros((16,), jnp.float32)
    iota16 = lax.iota(jnp.int32, 16)
    c32 = jnp.full((16,), 32, jnp.int32)
    c0 = jnp.zeros((16,), jnp.int32)
    chead = c0 + head
    slots = ((rows_a, adrow_a, sem_ra, sem_aa),
             (rows_b, adrow_b, sem_rb, sem_ab))
    ohslots = ((oh_a, ddiv_a, sem_oa), (oh_b, ddiv_b, sem_ob))
    oh_v = oh_a

    def zs(r, _):
      srows_v[r, pl.ds(0, 16)] = zvec
      srows_v[r, pl.ds(16, 16)] = zvec
      oh_a[r, :] = zvec
      oh_b[r, :] = zvec
      return 0

    lax.fori_loop(0, SUB, zs, 0)

    def zcp(k, _):
      c = sid + 16 * k

      @pl.when(c < nch)
      def _():
        pltpu.sync_copy(srows_v.at[pl.ds(0, drn)],
                        out_sp.at[pl.ds(c * drn, drn)])

      return 0

    lax.fori_loop(0, ndr, zcp, 0)

    def zcd(k, _):
      c = sid + 16 * k

      @pl.when(c < dch)
      def _():
        pltpu.sync_copy(oh_v, den_sp.at[pl.ds(c * SUB, SUB)])

      return 0

    lax.fori_loop(0, ndd, zcd, 0)
    plsc.subcore_barrier()

    coff = cid * n

    def chunk_body(ch, _):
      base = sid * et + ch * CE
      pltpu.sync_copy(srcp_hbm.at[pl.ds(base, CE)], src_v)
      pltpu.sync_copy(dst3_hbm.at[sid, pl.ds(ch * (CE // 128), CE // 128)],
                      dst_v)

      def rb(q, _):
        src_v[pl.ds(q * 16, 16)] = src_v[pl.ds(q * 16, 16)] + coff
        return 0

      lax.fori_loop(0, CE // 16, rb, 0)

      def issue(j, slot):
        rows_s, adrow_s, sem_r, sem_a = slot
        h1 = pltpu.async_copy(
            hsrc_hbm.at[src_v.at[pl.ds(j * SUB, SUB)]], rows_s, sem_r)
        h2 = pltpu.async_copy(adp_hbm.at[dst_v.at[j]], adrow_s, sem_a)
        return (h1, h2)

      def rz(j, oh_s):
        def body(l, _):
          rid = iota16 + l * 16
          d16 = dst_v[j, pl.ds(l * 16, 16)]
          dmod = jnp.bitwise_and(d16, 15)
          plsc.store_scatter(oh_s, [rid, dmod], zvec)
          return 0

        lax.fori_loop(0, SUB // 16, body, 0)

      pend = issue(0, slots[0])
      oh_pend = [None, None]
      for j in range(nsub):
        if j + 1 < nsub:
          nxt = issue(j + 1, slots[(j + 1) % 2])
        rows_s, adrow_s, _, _ = slots[j % 2]
        oh_s, ddiv_s, sem_o = ohslots[j % 2]
        if oh_pend[j % 2] is not None:
          oh_pend[j % 2].wait()
          rz(j - 2, oh_s)
        pend[0].wait()
        pend[1].wait()

        def ohb(l, _):
          rid = iota16 + l * 16
          asg = plsc.load_gather(rows_s, [rid, c32])
          adg = plsc.load_gather(adrow_s, [rid, chead])
          t0 = asg + adg
          al = jnp.maximum(t0, 0.2 * t0)
          t1 = mvec + adg
          sh = jnp.maximum(t1, 0.2 * t1)
          exv = jnp.exp(al - sh)
          gid = iota16 + (base + j * SUB + l * 16)
          exv = jnp.where(gid < e_real, exv, 0.0)
          ex_v[pl.ds(l * 16, 16)] = exv
          d16 = dst_v[j, pl.ds(l * 16, 16)]
          dmod = jnp.bitwise_and(d16, 15)
          ddiv = jnp.right_shift(d16, 4)
          ddiv_s[0, pl.ds(l * 16, 16)] = ddiv
          plsc.store_scatter(oh_s, [rid, dmod], exv)
          return 0

        lax.fori_loop(0, SUB // 16, ohb, 0)

        def scale(e2, _):
          exs = ex_v[pl.ds(e2, 16)][0]
          srows_v[e2, pl.ds(0, 16)] = rows_s[e2, pl.ds(0, 16)] * exs
          srows_v[e2, pl.ds(16, 16)] = rows_s[e2, pl.ds(16, 16)] * exs
          return 0

        lax.fori_loop(0, SUB, scale, 0, unroll=8)

        pltpu.sync_copy(srows_v, out_sp.at[dst_v.at[j]], add=True)
        oh_pend[j % 2] = pltpu.async_copy(
            oh_s, den_sp.at[ddiv_s.at[0]], sem_o, add=True)
        if j + 1 < nsub:
          pend = nxt

      for j in (nsub - 2, nsub - 1):
        q = j % 2
        if oh_pend[q] is not None:
          oh_pend[q].wait()
          rz(j, ohslots[q][0])
      return 0

    lax.fori_loop(0, n_chunks, chunk_body, 0)
    plsc.subcore_barrier()

    def drain(k, _):
      c = sid + 16 * k

      @pl.when(c < nch)
      def _():
        pltpu.sync_copy(out_sp.at[pl.ds(c * drn, drn)],
                        srows_v.at[pl.ds(0, drn)])
        pltpu.sync_copy(srows_v.at[pl.ds(0, drn)],
                        num_hbm.at[cid, pl.ds(c * drn, drn)])

      return 0

    lax.fori_loop(0, ndr, drain, 0)

    def draind(k, _):
      c = sid + 16 * k

      @pl.when(c < dch)
      def _():
        pltpu.sync_copy(den_sp.at[pl.ds(c * SUB, SUB)], oh_v)
        pltpu.sync_copy(oh_v, den_hbm.at[cid, pl.ds(c * SUB, SUB)])

      return 0

    lax.fori_loop(0, ndd, draind, 0)

  return sc_k(hsrc, srcp, dst3, adp, mrow)


def kernel(x, edge_index, batch, W_emb, b_emb, W1, att_src1, att_dst1, b1,
           W2, att_src2, att_dst2, b2, gamma1, beta1, gamma2, beta2,
           W_out, b_out):
  n = x.shape[0]
  e = edge_index.shape[1]
  g_groups = 64
  f32 = jnp.float32

  ep = ((e + 16 * CE - 1) // (16 * CE)) * (16 * CE)
  src = edge_index[0]
  dst = edge_index[1]
  srcp = jnp.concatenate([src, jnp.zeros((ep - e,), jnp.int32)])
  dstp = jnp.concatenate([dst, jnp.zeros((ep - e,), jnp.int32)])
  dst3 = dstp.reshape(16, (ep // 16) // 128, 128)
  batch2d = batch.reshape(n, 1)

  a1 = jnp.zeros((64, 2), f32)
  a1 = a1.at[0:32, 0].set(att_src1[0]).at[32:64, 1].set(att_src1[1])
  b1a = jnp.zeros((64, 2), f32)
  b1a = b1a.at[0:32, 0].set(att_dst1[0]).at[32:64, 1].set(att_dst1[1])
  a2 = att_src2.T
  b2a = att_dst2.T

  bemb2d = b_emb.reshape(1, 64)
  b1_2d = b1.reshape(1, 64)
  b2_2d = b2.reshape(1, 64)
  g1_2d = gamma1.reshape(1, 64)
  be1_2d = beta1.reshape(1, 64)
  g2_2d = gamma2.reshape(1, 64)
  be2_2d = beta2.reshape(1, 64)
  bout2d = b_out.reshape(1, 128)

  zpad15 = jnp.zeros((n, 15), f32)

  hw1, as1, ad1, m1 = _prep_call(True, x, W_emb, bemb2d, W1, a1, b1a, 2)
  hsrc1 = jnp.concatenate([
      jnp.concatenate([hw1[:, :32], as1[:, 0:1], zpad15], axis=1),
      jnp.concatenate([hw1[:, 32:], as1[:, 1:2], zpad15], axis=1),
  ], axis=0)
  adp1 = jnp.concatenate([ad1, jnp.zeros((n, 14), f32)], axis=1)
  mp1 = jnp.broadcast_to(m1.T, (2, 16))
  num1, den1 = _sc_edge_call(hsrc1, srcp, dst3, adp1, mp1,
                             n=n, e_real=e, head_is_core=True)
  num1f = jnp.concatenate([num1[0], num1[1]], axis=1)
  den1f = jnp.stack(
      [den1[0].reshape(-1)[:n], den1[1].reshape(-1)[:n]], axis=1)
  conv1, s1, q1 = _post_call(2, num1f, den1f, as1, ad1, m1, hw1, b1_2d)

  hw2, as2, ad2, m2 = _bnprep_call(conv1, s1, q1, g1_2d, be1_2d, W2, a2,
                                   b2a, 1)
  hsrc2 = jnp.concatenate([
      jnp.concatenate([hw2[:, :32], as2, zpad15], axis=1),
      jnp.concatenate([hw2[:, 32:], as2, zpad15], axis=1),
  ], axis=0)
  adp2 = jnp.concatenate([ad2, zpad15], axis=1)
  mp2 = jnp.broadcast_to(m2.T, (1, 16))
  num2, den2 = _sc_edge_call(hsrc2, srcp, dst3, adp2, mp2,
                             n=n, e_real=e, head_is_core=False)
  num2f = jnp.concatenate([num2[0], num2[1]], axis=1)
  den2f = den2[0].reshape(-1)[:n].reshape(n, 1)
  conv2, s2, q2 = _post_call(1, num2f, den2f, as2, ad2, m2, hw2, b2_2d)

  pmax, psum, pcnt = _pool_call(conv2, s2, q2, g2_2d, be2_2d, batch2d,
                                g_groups)
  return _final_call(pmax, psum, pcnt, W_out, b_out.reshape(1, 128))


# speedup vs baseline: 50.4799x; 1.0458x over previous
import functools

import jax
import jax.numpy as jnp
from jax import lax
from jax.experimental import pallas as pl
from jax.experimental.pallas import tpu as pltpu
from jax.experimental.pallas import tpu_sc as plsc

RB = 1000
CE = 1024
SUB = 128
NEG_INF = float("-inf")


def _elu(x):
  return jnp.where(x > 0, x, jnp.exp(jnp.minimum(x, 0.0)) - 1.0)


def _prep_kernel(embed, h_r, wemb_r, bemb_r, w_r, a_r, b_r,
                 hw_o, as_o, ad_o, m_o):
  i = pl.program_id(0)
  h = h_r[...]
  if embed:
    h = _elu(h @ wemb_r[...] + bemb_r[...])
  hw = h @ w_r[...]
  hw_o[...] = hw
  asb = hw @ a_r[...]
  adb = hw @ b_r[...]
  as_o[...] = asb
  ad_o[...] = adb

  @pl.when(i == 0)
  def _():
    m_o[...] = jnp.full_like(m_o[...], NEG_INF)

  m_o[...] = jnp.maximum(m_o[...], jnp.max(asb, axis=0, keepdims=True))


def _prep_call(embed, h_in, wemb, bemb, w, a, b, heads):
  n = h_in.shape[0]
  nb = n // RB
  cin = h_in.shape[1]
  full = lambda shp: pl.BlockSpec(shp, lambda i: (0, 0))
  return pl.pallas_call(
      functools.partial(_prep_kernel, embed),
      grid=(nb,),
      in_specs=[
          pl.BlockSpec((RB, cin), lambda i: (i, 0)),
          full(wemb.shape), full(bemb.shape), full(w.shape),
          full(a.shape), full(b.shape),
      ],
      out_specs=[
          pl.BlockSpec((RB, 64), lambda i: (i, 0)),
          pl.BlockSpec((RB, heads), lambda i: (i, 0)),
          pl.BlockSpec((RB, heads), lambda i: (i, 0)),
          pl.BlockSpec((1, heads), lambda i: (0, 0)),
      ],
      out_shape=[
          jax.ShapeDtypeStruct((n, 64), jnp.float32),
          jax.ShapeDtypeStruct((n, heads), jnp.float32),
          jax.ShapeDtypeStruct((n, heads), jnp.float32),
          jax.ShapeDtypeStruct((1, heads), jnp.float32),
      ],
  )(h_in, wemb, bemb, w, a, b)


def _post_kernel(heads, n, num_r, den_r, as_r, ad_r, m_r, hw_r, b_r,
                 conv_o, ssum_o, ssq_o):
  i = pl.program_id(0)
  c = 64 // heads
  asb = as_r[...]
  adb = ad_r[...]
  m = m_r[...]
  t = asb + adb
  al = jnp.maximum(t, 0.2 * t)
  t2 = m + adb
  d2 = jnp.maximum(t2, 0.2 * t2)
  sex = jnp.exp(al - d2)
  den = den_r[...] + sex
  hw = hw_r[...]
  num = num_r[...]
  parts = []
  for h in range(heads):
    nh = num[:, h * c:(h + 1) * c] + sex[:, h:h + 1] * hw[:, h * c:(h + 1) * c]
    parts.append(nh / (den[:, h:h + 1] + 1e-16))
  conv = (jnp.concatenate(parts, axis=1) if heads > 1 else parts[0]) + b_r[...]
  conv_o[...] = conv

  @pl.when(i == 0)
  def _():
    ssum_o[...] = jnp.zeros_like(ssum_o[...])
    ssq_o[...] = jnp.zeros_like(ssq_o[...])

  ssum_o[...] += jnp.sum(conv, axis=0, keepdims=True)
  ssq_o[...] += jnp.sum(conv * conv, axis=0, keepdims=True)


def _post_call(heads, num, den, a_s, a_d, m, hw, bias):
  n = num.shape[0]
  nb = n // RB
  full = lambda shp: pl.BlockSpec(shp, lambda i: (0, 0))
  return pl.pallas_call(
      functools.partial(_post_kernel, heads, n),
      grid=(nb,),
      in_specs=[
          pl.BlockSpec((RB, 64), lambda i: (i, 0)),
          pl.BlockSpec((RB, heads), lambda i: (i, 0)),
          pl.BlockSpec((RB, heads), lambda i: (i, 0)),
          pl.BlockSpec((RB, heads), lambda i: (i, 0)),
          full((1, heads)),
          pl.BlockSpec((RB, 64), lambda i: (i, 0)),
          full((1, 64)),
      ],
      out_specs=[
          pl.BlockSpec((RB, 64), lambda i: (i, 0)),
          full((1, 64)), full((1, 64)),
      ],
      out_shape=[
          jax.ShapeDtypeStruct((n, 64), jnp.float32),
          jax.ShapeDtypeStruct((1, 64), jnp.float32),
          jax.ShapeDtypeStruct((1, 64), jnp.float32),
      ],
  )(num, den, a_s, a_d, m, hw, bias)


def _bnprep_kernel(n, conv_r, ssum_r, ssq_r, g_r, be_r, w_r, a_r, b_r,
                   hw_o, as_o, ad_o, m_o):
  i = pl.program_id(0)
  mu = ssum_r[...] / n
  var = ssq_r[...] / n - mu * mu
  y = (conv_r[...] - mu) / jnp.sqrt(var + 1e-5) * g_r[...] + be_r[...]
  h = _elu(y)
  hw = h @ w_r[...]
  hw_o[...] = hw
  asb = hw @ a_r[...]
  adb = hw @ b_r[...]
  as_o[...] = asb
  ad_o[...] = adb

  @pl.when(i == 0)
  def _():
    m_o[...] = jnp.full_like(m_o[...], NEG_INF)

  m_o[...] = jnp.maximum(m_o[...], jnp.max(asb, axis=0, keepdims=True))


def _bnprep_call(conv, ssum, ssq, gamma, beta, w, a, b, heads):
  n = conv.shape[0]
  nb = n // RB
  full = lambda shp: pl.BlockSpec(shp, lambda i: (0, 0))
  return pl.pallas_call(
      functools.partial(_bnprep_kernel, n),
      grid=(nb,),
      in_specs=[
          pl.BlockSpec((RB, 64), lambda i: (i, 0)),
          full((1, 64)), full((1, 64)), full((1, 64)), full((1, 64)),
          full((64, 64)), full((64, heads)), full((64, heads)),
      ],
      out_specs=[
          pl.BlockSpec((RB, 64), lambda i: (i, 0)),
          pl.BlockSpec((RB, heads), lambda i: (i, 0)),
          pl.BlockSpec((RB, heads), lambda i: (i, 0)),
          pl.BlockSpec((1, heads), lambda i: (0, 0)),
      ],
      out_shape=[
          jax.ShapeDtypeStruct((n, 64), jnp.float32),
          jax.ShapeDtypeStruct((n, heads), jnp.float32),
          jax.ShapeDtypeStruct((n, heads), jnp.float32),
          jax.ShapeDtypeStruct((1, heads), jnp.float32),
      ],
  )(conv, ssum, ssq, gamma, beta, w, a, b)


def _pool_kernel(n, g_groups, conv_r, ssum_r, ssq_r, g_r, be_r, bt_r,
                 pmax_o, psum_o, pcnt_o):
  i = pl.program_id(0)
  mu = ssum_r[...] / n
  var = ssq_r[...] / n - mu * mu
  y = (conv_r[...] - mu) / jnp.sqrt(var + 1e-5) * g_r[...] + be_r[...]
  h = _elu(y)
  bt = bt_r[...]

  @pl.when(i == 0)
  def _():
    pmax_o[...] = jnp.full_like(pmax_o[...], NEG_INF)
    psum_o[...] = jnp.zeros_like(psum_o[...])
    pcnt_o[...] = jnp.zeros_like(pcnt_o[...])

  g0 = bt[0, 0]
  g1 = bt[RB - 1, 0]

  def body(g, _):
    mask = bt == g
    hm = jnp.where(mask, h, NEG_INF)
    gmax = jnp.max(hm, axis=0, keepdims=True)
    hs = jnp.where(mask, h, 0.0)
    gsum = jnp.sum(hs, axis=0, keepdims=True)
    gcnt = jnp.sum(jnp.where(mask, 1.0, 0.0))
    pmax_o[pl.ds(g, 1), :] = jnp.maximum(pmax_o[pl.ds(g, 1), :], gmax)
    psum_o[pl.ds(g, 1), :] = psum_o[pl.ds(g, 1), :] + gsum
    pcnt_o[pl.ds(g, 1), :] = pcnt_o[pl.ds(g, 1), :] + gcnt
    return 0

  lax.fori_loop(g0, g1 + 1, body, 0)


def _pool_call(conv, ssum, ssq, gamma, beta, batch2d, g_groups):
  n = conv.shape[0]
  nb = n // RB
  full = lambda shp: pl.BlockSpec(shp, lambda i: (0, 0))
  return pl.pallas_call(
      functools.partial(_pool_kernel, n, g_groups),
      grid=(nb,),
      in_specs=[
          pl.BlockSpec((RB, 64), lambda i: (i, 0)),
          full((1, 64)), full((1, 64)), full((1, 64)), full((1, 64)),
          pl.BlockSpec((RB, 1), lambda i: (i, 0)),
      ],
      out_specs=[
          full((g_groups, 64)), full((g_groups, 64)), full((g_groups, 64)),
      ],
      out_shape=[
          jax.ShapeDtypeStruct((g_groups, 64), jnp.float32),
          jax.ShapeDtypeStruct((g_groups, 64), jnp.float32),
          jax.ShapeDtypeStruct((g_groups, 64), jnp.float32),
      ],
  )(conv, ssum, ssq, gamma, beta, batch2d)


def _final_kernel(pmax_r, psum_r, pcnt_r, w_r, b_r, out_o):
  pmax = pmax_r[...]
  xmax = jnp.where(pmax == NEG_INF, 0.0, pmax)
  xmean = psum_r[...] / jnp.maximum(pcnt_r[...], 1.0)
  comb = jnp.concatenate([xmax, xmean], axis=1)
  out_o[...] = comb @ w_r[...] + b_r[...]


def _final_call(pmax, psum, pcnt, w_out, b_out):
  g = pmax.shape[0]
  return pl.pallas_call(
      _final_kernel,
      out_shape=jax.ShapeDtypeStruct((g, 128), jnp.float32),
  )(pmax, psum, pcnt, w_out, b_out)


def _sc_edge_call(hsrc, srcp, dst3, adp, mrow, *, n, e_real, head_is_core):
  ep = srcp.shape[0]
  et = ep // 16
  n_chunks = et // CE
  nsub = CE // SUB
  drn = 80
  nch = n // drn
  ndr = (nch + 15) // 16
  nden = ((n // 16 + 127) // 128) * 128
  dch = nden // SUB
  ndd = (dch + 15) // 16
  mesh = plsc.VectorSubcoreMesh(core_axis_name="c", subcore_axis_name="s")

  @functools.partial(
      pl.kernel,
      mesh=mesh,
      compiler_params=pltpu.CompilerParams(
          needs_layout_passes=False, use_tc_tiling_on_sc=False),
      out_type=[
          jax.ShapeDtypeStruct((2, n, 32), jnp.float32),
          jax.ShapeDtypeStruct((2, nden, 16), jnp.float32),
      ],
      scratch_types=[
          pltpu.VMEM((CE,), jnp.int32),
          pltpu.VMEM((CE // 128, 128), jnp.int32),
          pltpu.VMEM((SUB + 16,), jnp.float32),
          pltpu.VMEM((SUB, 48), jnp.float32),
          pltpu.VMEM((SUB, 48), jnp.float32),
          pltpu.VMEM((SUB, 16), jnp.float32),
          pltpu.VMEM((SUB, 16), jnp.float32),
          pltpu.VMEM((SUB, 32), jnp.float32),
          pltpu.VMEM((SUB, 16), jnp.float32),
          pltpu.VMEM((SUB, 16), jnp.float32),
          pltpu.VMEM((1, 128), jnp.int32),
          pltpu.VMEM((1, 128), jnp.int32),
          pltpu.VMEM((16,), jnp.float32),
          pltpu.SemaphoreType.DMA,
          pltpu.SemaphoreType.DMA,
          pltpu.SemaphoreType.DMA,
          pltpu.SemaphoreType.DMA,
          pltpu.SemaphoreType.DMA,
          pltpu.SemaphoreType.DMA,
          pltpu.SemaphoreType.DMA,
          pltpu.VMEM_SHARED((n, 32), jnp.float32),
          pltpu.VMEM_SHARED((nden, 16), jnp.float32),
      ],
  )
  def sc_k(hsrc_hbm, srcp_hbm, dst3_hbm, adp_hbm, m_hbm,
           num_hbm, den_hbm,
           src_v, dst_v, ex_v, rows_a, rows_b, adrow_a, adrow_b,
           srows_v, oh_a, oh_b, ddiv_a, ddiv_b, m_v,
           sem_ra, sem_rb, sem_aa, sem_ab, sem_oa, sem_ob, sem_sr,
           out_sp, den_sp):
    cid = lax.axis_index("c")
    sid = lax.axis_index("s")
    head = cid if head_is_core else 0
    pltpu.sync_copy(m_hbm.at[head], m_v)
    mvec = m_v[...]
    zvec = jnp.zeros((16,), jnp.float32)
    iota16 = lax.iota(jnp.int32, 16)
    c32 = jnp.full((16,), 32, jnp.int32)
    c0 = jnp.zeros((16,), jnp.int32)
    chead = c0 + head
    slots = ((rows_a, adrow_a, sem_ra, sem_aa),
             (rows_b, adrow_b, sem_rb, sem_ab))
    ohslots = ((oh_a, ddiv_a, sem_oa), (oh_b, ddiv_b, sem_ob))
    oh_v = oh_a

    def zs(r, _):
      srows_v[r, pl.ds(0, 16)] = zvec
      srows_v[r, pl.ds(16, 16)] = zvec
      oh_a[r, :] = zvec
      oh_b[r, :] = zvec
      return 0

    lax.fori_loop(0, SUB, zs, 0)

    def zcp(k, _):
      c = sid + 16 * k

      @pl.when(c < nch)
      def _():
        pltpu.sync_copy(srows_v.at[pl.ds(0, drn)],
                        out_sp.at[pl.ds(c * drn, drn)])

      return 0

    lax.fori_loop(0, ndr, zcp, 0)

    def zcd(k, _):
      c = sid + 16 * k

      @pl.when(c < dch)
      def _():
        pltpu.sync_copy(oh_v, den_sp.at[pl.ds(c * SUB, SUB)])

      return 0

    lax.fori_loop(0, ndd, zcd, 0)
    plsc.subcore_barrier()

    coff = cid * n

    def chunk_body(ch, _):
      base = sid * et + ch * CE
      pltpu.sync_copy(srcp_hbm.at[pl.ds(base, CE)], src_v)
      pltpu.sync_copy(dst3_hbm.at[sid, pl.ds(ch * (CE // 128), CE // 128)],
                      dst_v)

      def rb(q, _):
        src_v[pl.ds(q * 16, 16)] = src_v[pl.ds(q * 16, 16)] + coff
        return 0

      lax.fori_loop(0, CE // 16, rb, 0)

      def issue(j, slot):
        rows_s, adrow_s, sem_r, sem_a = slot
        h1 = pltpu.async_copy(
            hsrc_hbm.at[src_v.at[pl.ds(j * SUB, SUB)]], rows_s, sem_r)
        h2 = pltpu.async_copy(adp_hbm.at[dst_v.at[j]], adrow_s, sem_a)
        return (h1, h2)

      def rz(j, oh_s):
        def body(l, _):
          rid = iota16 + l * 16
          d16 = dst_v[j, pl.ds(l * 16, 16)]
          dmod = jnp.bitwise_and(d16, 15)
          plsc.store_scatter(oh_s, [rid, dmod], zvec)
          return 0

        lax.fori_loop(0, SUB // 16, body, 0)

      pend = issue(0, slots[0])
      oh_pend = [None, None]
      sr_pend = None
      for j in range(nsub):
        if j + 1 < nsub:
          nxt = issue(j + 1, slots[(j + 1) % 2])
        rows_s, adrow_s, _, _ = slots[j % 2]
        oh_s, ddiv_s, sem_o = ohslots[j % 2]
        if oh_pend[j % 2] is not None:
          oh_pend[j % 2].wait()
          rz(j - 2, oh_s)
        pend[0].wait()
        pend[1].wait()

        def ohb(l, _):
          rid = iota16 + l * 16
          asg = plsc.load_gather(rows_s, [rid, c32])
          adg = plsc.load_gather(adrow_s, [rid, chead])
          t0 = asg + adg
          al = jnp.maximum(t0, 0.2 * t0)
          t1 = mvec + adg
          sh = jnp.maximum(t1, 0.2 * t1)
          exv = jnp.exp(al - sh)
          gid = iota16 + (base + j * SUB + l * 16)
          exv = jnp.where(gid < e_real, exv, 0.0)
          ex_v[pl.ds(l * 16, 16)] = exv
          d16 = dst_v[j, pl.ds(l * 16, 16)]
          dmod = jnp.bitwise_and(d16, 15)
          ddiv = jnp.right_shift(d16, 4)
          ddiv_s[0, pl.ds(l * 16, 16)] = ddiv
          plsc.store_scatter(oh_s, [rid, dmod], exv)
          return 0

        lax.fori_loop(0, SUB // 16, ohb, 0, unroll=2)

        if sr_pend is not None:
          sr_pend.wait()

        def scale(e2, _):
          exs = ex_v[pl.ds(e2, 16)][0]
          srows_v[e2, pl.ds(0, 16)] = rows_s[e2, pl.ds(0, 16)] * exs
          srows_v[e2, pl.ds(16, 16)] = rows_s[e2, pl.ds(16, 16)] * exs
          return 0

        lax.fori_loop(0, SUB, scale, 0, unroll=8)

        sr_pend = pltpu.async_copy(
            srows_v, out_sp.at[dst_v.at[j]], sem_sr, add=True)
        oh_pend[j % 2] = pltpu.async_copy(
            oh_s, den_sp.at[ddiv_s.at[0]], sem_o, add=True)
        if j + 1 < nsub:
          pend = nxt

      sr_pend.wait()
      for j in (nsub - 2, nsub - 1):
        q = j % 2
        if oh_pend[q] is not None:
          oh_pend[q].wait()
          rz(j, ohslots[q][0])
      return 0

    lax.fori_loop(0, n_chunks, chunk_body, 0)
    plsc.subcore_barrier()

    def drain(k, _):
      c = sid + 16 * k

      @pl.when(c < nch)
      def _():
        pltpu.sync_copy(out_sp.at[pl.ds(c * drn, drn)],
                        srows_v.at[pl.ds(0, drn)])
        pltpu.sync_copy(srows_v.at[pl.ds(0, drn)],
                        num_hbm.at[cid, pl.ds(c * drn, drn)])

      return 0

    lax.fori_loop(0, ndr, drain, 0)

    def draind(k, _):
      c = sid + 16 * k

      @pl.when(c < dch)
      def _():
        pltpu.sync_copy(den_sp.at[pl.ds(c * SUB, SUB)], oh_v)
        pltpu.sync_copy(oh_v, den_hbm.at[cid, pl.ds(c * SUB, SUB)])

      return 0

    lax.fori_loop(0, ndd, draind, 0)

  return sc_k(hsrc, srcp, dst3, adp, mrow)


def kernel(x, edge_index, batch, W_emb, b_emb, W1, att_src1, att_dst1, b1,
           W2, att_src2, att_dst2, b2, gamma1, beta1, gamma2, beta2,
           W_out, b_out):
  n = x.shape[0]
  e = edge_index.shape[1]
  g_groups = 64
  f32 = jnp.float32

  ep = ((e + 16 * CE - 1) // (16 * CE)) * (16 * CE)
  src = edge_index[0]
  dst = edge_index[1]
  srcp = jnp.concatenate([src, jnp.zeros((ep - e,), jnp.int32)])
  dstp = jnp.concatenate([dst, jnp.zeros((ep - e,), jnp.int32)])
  dst3 = dstp.reshape(16, (ep // 16) // 128, 128)
  batch2d = batch.reshape(n, 1)

  a1 = jnp.zeros((64, 2), f32)
  a1 = a1.at[0:32, 0].set(att_src1[0]).at[32:64, 1].set(att_src1[1])
  b1a = jnp.zeros((64, 2), f32)
  b1a = b1a.at[0:32, 0].set(att_dst1[0]).at[32:64, 1].set(att_dst1[1])
  a2 = att_src2.T
  b2a = att_dst2.T

  bemb2d = b_emb.reshape(1, 64)
  b1_2d = b1.reshape(1, 64)
  b2_2d = b2.reshape(1, 64)
  g1_2d = gamma1.reshape(1, 64)
  be1_2d = beta1.reshape(1, 64)
  g2_2d = gamma2.reshape(1, 64)
  be2_2d = beta2.reshape(1, 64)
  bout2d = b_out.reshape(1, 128)

  zpad15 = jnp.zeros((n, 15), f32)

  hw1, as1, ad1, m1 = _prep_call(True, x, W_emb, bemb2d, W1, a1, b1a, 2)
  hsrc1 = jnp.concatenate([
      jnp.concatenate([hw1[:, :32], as1[:, 0:1], zpad15], axis=1),
      jnp.concatenate([hw1[:, 32:], as1[:, 1:2], zpad15], axis=1),
  ], axis=0)
  adp1 = jnp.concatenate([ad1, jnp.zeros((n, 14), f32)], axis=1)
  mp1 = jnp.broadcast_to(m1.T, (2, 16))
  num1, den1 = _sc_edge_call(hsrc1, srcp, dst3, adp1, mp1,
                             n=n, e_real=e, head_is_core=True)
  num1f = jnp.concatenate([num1[0], num1[1]], axis=1)
  den1f = jnp.stack(
      [den1[0].reshape(-1)[:n], den1[1].reshape(-1)[:n]], axis=1)
  conv1, s1, q1 = _post_call(2, num1f, den1f, as1, ad1, m1, hw1, b1_2d)

  hw2, as2, ad2, m2 = _bnprep_call(conv1, s1, q1, g1_2d, be1_2d, W2, a2,
                                   b2a, 1)
  hsrc2 = jnp.concatenate([
      jnp.concatenate([hw2[:, :32], as2, zpad15], axis=1),
      jnp.concatenate([hw2[:, 32:], as2, zpad15], axis=1),
  ], axis=0)
  adp2 = jnp.concatenate([ad2, zpad15], axis=1)
  mp2 = jnp.broadcast_to(m2.T, (1, 16))
  num2, den2 = _sc_edge_call(hsrc2, srcp, dst3, adp2, mp2,
                             n=n, e_real=e, head_is_core=False)
  num2f = jnp.concatenate([num2[0], num2[1]], axis=1)
  den2f = den2[0].reshape(-1)[:n].reshape(n, 1)
  conv2, s2, q2 = _post_call(1, num2f, den2f, as2, ad2, m2, hw2, b2_2d)

  pmax, psum, pcnt = _pool_call(conv2, s2, q2, g2_2d, be2_2d, batch2d,
                                g_groups)
  return _final_call(pmax, psum, pcnt, W_out, b_out.reshape(1, 128))


# speedup vs baseline: 52.4595x; 1.0392x over previous
import functools

import jax
import jax.numpy as jnp
from jax import lax
from jax.experimental import pallas as pl
from jax.experimental.pallas import tpu as pltpu
from jax.experimental.pallas import tpu_sc as plsc

RB = 1000
CE = 1024
SUB = 128
NEG_INF = float("-inf")


def _elu(x):
  return jnp.where(x > 0, x, jnp.exp(jnp.minimum(x, 0.0)) - 1.0)


def _prep_kernel(embed, h_r, wemb_r, bemb_r, w_r, a_r, b_r,
                 hw_o, as_o, ad_o, m_o):
  i = pl.program_id(0)
  h = h_r[...]
  if embed:
    h = _elu(h @ wemb_r[...] + bemb_r[...])
  hw = h @ w_r[...]
  hw_o[...] = hw
  asb = hw @ a_r[...]
  adb = hw @ b_r[...]
  as_o[...] = asb
  ad_o[...] = adb

  @pl.when(i == 0)
  def _():
    m_o[...] = jnp.full_like(m_o[...], NEG_INF)

  m_o[...] = jnp.maximum(m_o[...], jnp.max(asb, axis=0, keepdims=True))


def _prep_call(embed, h_in, wemb, bemb, w, a, b, heads):
  n = h_in.shape[0]
  nb = n // RB
  cin = h_in.shape[1]
  full = lambda shp: pl.BlockSpec(shp, lambda i: (0, 0))
  return pl.pallas_call(
      functools.partial(_prep_kernel, embed),
      grid=(nb,),
      in_specs=[
          pl.BlockSpec((RB, cin), lambda i: (i, 0)),
          full(wemb.shape), full(bemb.shape), full(w.shape),
          full(a.shape), full(b.shape),
      ],
      out_specs=[
          pl.BlockSpec((RB, 64), lambda i: (i, 0)),
          pl.BlockSpec((RB, heads), lambda i: (i, 0)),
          pl.BlockSpec((RB, heads), lambda i: (i, 0)),
          pl.BlockSpec((1, heads), lambda i: (0, 0)),
      ],
      out_shape=[
          jax.ShapeDtypeStruct((n, 64), jnp.float32),
          jax.ShapeDtypeStruct((n, heads), jnp.float32),
          jax.ShapeDtypeStruct((n, heads), jnp.float32),
          jax.ShapeDtypeStruct((1, heads), jnp.float32),
      ],
  )(h_in, wemb, bemb, w, a, b)


def _post_kernel(heads, n, num_lo_r, num_hi_r, den_r, as_r, ad_r, m_r,
                 hw_r, b_r, conv_o, ssum_o, ssq_o):
  i = pl.program_id(0)
  c = 64 // heads
  asb = as_r[...]
  adb = ad_r[...]
  m = m_r[...]
  t = asb + adb
  al = jnp.maximum(t, 0.2 * t)
  t2 = m + adb
  d2 = jnp.maximum(t2, 0.2 * t2)
  sex = jnp.exp(al - d2)
  den = den_r[...] + sex
  hw = hw_r[...]
  num = jnp.concatenate([num_lo_r[0], num_hi_r[0]], axis=1)
  parts = []
  for h in range(heads):
    nh = num[:, h * c:(h + 1) * c] + sex[:, h:h + 1] * hw[:, h * c:(h + 1) * c]
    parts.append(nh / (den[:, h:h + 1] + 1e-16))
  conv = (jnp.concatenate(parts, axis=1) if heads > 1 else parts[0]) + b_r[...]
  conv_o[...] = conv

  @pl.when(i == 0)
  def _():
    ssum_o[...] = jnp.zeros_like(ssum_o[...])
    ssq_o[...] = jnp.zeros_like(ssq_o[...])

  ssum_o[...] += jnp.sum(conv, axis=0, keepdims=True)
  ssq_o[...] += jnp.sum(conv * conv, axis=0, keepdims=True)


def _post_call(heads, num3, den, a_s, a_d, m, hw, bias):
  n = num3.shape[1]
  nb = n // RB
  full = lambda shp: pl.BlockSpec(shp, lambda i: (0, 0))
  return pl.pallas_call(
      functools.partial(_post_kernel, heads, n),
      grid=(nb,),
      in_specs=[
          pl.BlockSpec((1, RB, 32), lambda i: (0, i, 0)),
          pl.BlockSpec((1, RB, 32), lambda i: (1, i, 0)),
          pl.BlockSpec((RB, heads), lambda i: (i, 0)),
          pl.BlockSpec((RB, heads), lambda i: (i, 0)),
          pl.BlockSpec((RB, heads), lambda i: (i, 0)),
          full((1, heads)),
          pl.BlockSpec((RB, 64), lambda i: (i, 0)),
          full((1, 64)),
      ],
      out_specs=[
          pl.BlockSpec((RB, 64), lambda i: (i, 0)),
          full((1, 64)), full((1, 64)),
      ],
      out_shape=[
          jax.ShapeDtypeStruct((n, 64), jnp.float32),
          jax.ShapeDtypeStruct((1, 64), jnp.float32),
          jax.ShapeDtypeStruct((1, 64), jnp.float32),
      ],
  )(num3, num3, den, a_s, a_d, m, hw, bias)


def _bnprep_kernel(n, conv_r, ssum_r, ssq_r, g_r, be_r, w_r, a_r, b_r,
                   hw_o, as_o, ad_o, m_o):
  i = pl.program_id(0)
  mu = ssum_r[...] / n
  var = ssq_r[...] / n - mu * mu
  y = (conv_r[...] - mu) / jnp.sqrt(var + 1e-5) * g_r[...] + be_r[...]
  h = _elu(y)
  hw = h @ w_r[...]
  hw_o[...] = hw
  asb = hw @ a_r[...]
  adb = hw @ b_r[...]
  as_o[...] = asb
  ad_o[...] = adb

  @pl.when(i == 0)
  def _():
    m_o[...] = jnp.full_like(m_o[...], NEG_INF)

  m_o[...] = jnp.maximum(m_o[...], jnp.max(asb, axis=0, keepdims=True))


def _bnprep_call(conv, ssum, ssq, gamma, beta, w, a, b, heads):
  n = conv.shape[0]
  nb = n // RB
  full = lambda shp: pl.BlockSpec(shp, lambda i: (0, 0))
  return pl.pallas_call(
      functools.partial(_bnprep_kernel, n),
      grid=(nb,),
      in_specs=[
          pl.BlockSpec((RB, 64), lambda i: (i, 0)),
          full((1, 64)), full((1, 64)), full((1, 64)), full((1, 64)),
          full((64, 64)), full((64, heads)), full((64, heads)),
      ],
      out_specs=[
          pl.BlockSpec((RB, 64), lambda i: (i, 0)),
          pl.BlockSpec((RB, heads), lambda i: (i, 0)),
          pl.BlockSpec((RB, heads), lambda i: (i, 0)),
          pl.BlockSpec((1, heads), lambda i: (0, 0)),
      ],
      out_shape=[
          jax.ShapeDtypeStruct((n, 64), jnp.float32),
          jax.ShapeDtypeStruct((n, heads), jnp.float32),
          jax.ShapeDtypeStruct((n, heads), jnp.float32),
          jax.ShapeDtypeStruct((1, heads), jnp.float32),
      ],
  )(conv, ssum, ssq, gamma, beta, w, a, b)


def _pool_kernel(n, g_groups, conv_r, ssum_r, ssq_r, g_r, be_r, bt_r,
                 pmax_o, psum_o, pcnt_o):
  i = pl.program_id(0)
  mu = ssum_r[...] / n
  var = ssq_r[...] / n - mu * mu
  y = (conv_r[...] - mu) / jnp.sqrt(var + 1e-5) * g_r[...] + be_r[...]
  h = _elu(y)
  bt = bt_r[...]

  @pl.when(i == 0)
  def _():
    pmax_o[...] = jnp.full_like(pmax_o[...], NEG_INF)
    psum_o[...] = jnp.zeros_like(psum_o[...])
    pcnt_o[...] = jnp.zeros_like(pcnt_o[...])

  g0 = bt[0, 0]
  g1 = bt[RB - 1, 0]

  def body(g, _):
    mask = bt == g
    hm = jnp.where(mask, h, NEG_INF)
    gmax = jnp.max(hm, axis=0, keepdims=True)
    hs = jnp.where(mask, h, 0.0)
    gsum = jnp.sum(hs, axis=0, keepdims=True)
    gcnt = jnp.sum(jnp.where(mask, 1.0, 0.0))
    pmax_o[pl.ds(g, 1), :] = jnp.maximum(pmax_o[pl.ds(g, 1), :], gmax)
    psum_o[pl.ds(g, 1), :] = psum_o[pl.ds(g, 1), :] + gsum
    pcnt_o[pl.ds(g, 1), :] = pcnt_o[pl.ds(g, 1), :] + gcnt
    return 0

  lax.fori_loop(g0, g1 + 1, body, 0)


def _pool_call(conv, ssum, ssq, gamma, beta, batch2d, g_groups):
  n = conv.shape[0]
  nb = n // RB
  full = lambda shp: pl.BlockSpec(shp, lambda i: (0, 0))
  return pl.pallas_call(
      functools.partial(_pool_kernel, n, g_groups),
      grid=(nb,),
      in_specs=[
          pl.BlockSpec((RB, 64), lambda i: (i, 0)),
          full((1, 64)), full((1, 64)), full((1, 64)), full((1, 64)),
          pl.BlockSpec((RB, 1), lambda i: (i, 0)),
      ],
      out_specs=[
          full((g_groups, 64)), full((g_groups, 64)), full((g_groups, 64)),
      ],
      out_shape=[
          jax.ShapeDtypeStruct((g_groups, 64), jnp.float32),
          jax.ShapeDtypeStruct((g_groups, 64), jnp.float32),
          jax.ShapeDtypeStruct((g_groups, 64), jnp.float32),
      ],
  )(conv, ssum, ssq, gamma, beta, batch2d)


def _final_kernel(pmax_r, psum_r, pcnt_r, w_r, b_r, out_o):
  pmax = pmax_r[...]
  xmax = jnp.where(pmax == NEG_INF, 0.0, pmax)
  xmean = psum_r[...] / jnp.maximum(pcnt_r[...], 1.0)
  comb = jnp.concatenate([xmax, xmean], axis=1)
  out_o[...] = comb @ w_r[...] + b_r[...]


def _final_call(pmax, psum, pcnt, w_out, b_out):
  g = pmax.shape[0]
  return pl.pallas_call(
      _final_kernel,
      out_shape=jax.ShapeDtypeStruct((g, 128), jnp.float32),
  )(pmax, psum, pcnt, w_out, b_out)


def _sc_edge_call(hsrc, srcp, dst3, adp, mrow, *, n, e_real, head_is_core):
  ep = srcp.shape[0]
  et = ep // 16
  n_chunks = et // CE
  nsub = CE // SUB
  drn = 80
  nch = n // drn
  ndr = (nch + 15) // 16
  nden = ((n // 16 + 127) // 128) * 128
  dch = nden // SUB
  ndd = (dch + 15) // 16
  mesh = plsc.VectorSubcoreMesh(core_axis_name="c", subcore_axis_name="s")

  @functools.partial(
      pl.kernel,
      mesh=mesh,
      compiler_params=pltpu.CompilerParams(
          needs_layout_passes=False, use_tc_tiling_on_sc=False),
      out_type=[
          jax.ShapeDtypeStruct((2, n, 32), jnp.float32),
          jax.ShapeDtypeStruct((2, nden, 16), jnp.float32),
      ],
      scratch_types=[
          pltpu.VMEM((CE,), jnp.int32),
          pltpu.VMEM((CE // 128, 128), jnp.int32),
          pltpu.VMEM((SUB + 16,), jnp.float32),
          pltpu.VMEM((SUB, 48), jnp.float32),
          pltpu.VMEM((SUB, 48), jnp.float32),
          pltpu.VMEM((SUB, 16), jnp.float32),
          pltpu.VMEM((SUB, 16), jnp.float32),
          pltpu.VMEM((SUB, 32), jnp.float32),
          pltpu.VMEM((SUB, 16), jnp.float32),
          pltpu.VMEM((SUB, 16), jnp.float32),
          pltpu.VMEM((1, 128), jnp.int32),
          pltpu.VMEM((1, 128), jnp.int32),
          pltpu.VMEM((16,), jnp.float32),
          pltpu.SemaphoreType.DMA,
          pltpu.SemaphoreType.DMA,
          pltpu.SemaphoreType.DMA,
          pltpu.SemaphoreType.DMA,
          pltpu.SemaphoreType.DMA,
          pltpu.SemaphoreType.DMA,
          pltpu.SemaphoreType.DMA,
          pltpu.VMEM_SHARED((n, 32), jnp.float32),
          pltpu.VMEM_SHARED((nden, 16), jnp.float32),
      ],
  )
  def sc_k(hsrc_hbm, srcp_hbm, dst3_hbm, adp_hbm, m_hbm,
           num_hbm, den_hbm,
           src_v, dst_v, ex_v, rows_a, rows_b, adrow_a, adrow_b,
           srows_v, oh_a, oh_b, ddiv_a, ddiv_b, m_v,
           sem_ra, sem_rb, sem_aa, sem_ab, sem_oa, sem_ob, sem_sr,
           out_sp, den_sp):
    cid = lax.axis_index("c")
    sid = lax.axis_index("s")
    head = cid if head_is_core else 0
    pltpu.sync_copy(m_hbm.at[head], m_v)
    mvec = m_v[...]
    zvec = jnp.zeros((16,), jnp.float32)
    iota16 = lax.iota(jnp.int32, 16)
    c32 = jnp.full((16,), 32, jnp.int32)
    c0 = jnp.zeros((16,), jnp.int32)
    chead = c0 + head
    slots = ((rows_a, adrow_a, sem_ra, sem_aa),
             (rows_b, adrow_b, sem_rb, sem_ab))
    ohslots = ((oh_a, ddiv_a, sem_oa), (oh_b, ddiv_b, sem_ob))
    oh_v = oh_a

    def zs(r, _):
      srows_v[r, pl.ds(0, 16)] = zvec
      srows_v[r, pl.ds(16, 16)] = zvec
      oh_a[r, :] = zvec
      oh_b[r, :] = zvec
      return 0

    lax.fori_loop(0, SUB, zs, 0)

    def zcp(k, _):
      c = sid + 16 * k

      @pl.when(c < nch)
      def _():
        pltpu.sync_copy(srows_v.at[pl.ds(0, drn)],
                        out_sp.at[pl.ds(c * drn, drn)])

      return 0

    lax.fori_loop(0, ndr, zcp, 0)

    def zcd(k, _):
      c = sid + 16 * k

      @pl.when(c < dch)
      def _():
        pltpu.sync_copy(oh_v, den_sp.at[pl.ds(c * SUB, SUB)])

      return 0

    lax.fori_loop(0, ndd, zcd, 0)
    plsc.subcore_barrier()

    coff = cid * n

    def chunk_body(ch, _):
      base = sid * et + ch * CE
      pltpu.sync_copy(srcp_hbm.at[pl.ds(base, CE)], src_v)
      pltpu.sync_copy(dst3_hbm.at[sid, pl.ds(ch * (CE // 128), CE // 128)],
                      dst_v)

      def rb(q, _):
        src_v[pl.ds(q * 16, 16)] = src_v[pl.ds(q * 16, 16)] + coff
        return 0

      lax.fori_loop(0, CE // 16, rb, 0)

      def issue(j, slot):
        rows_s, adrow_s, sem_r, sem_a = slot
        h1 = pltpu.async_copy(
            hsrc_hbm.at[src_v.at[pl.ds(j * SUB, SUB)]], rows_s, sem_r)
        h2 = pltpu.async_copy(adp_hbm.at[dst_v.at[j]], adrow_s, sem_a)
        return (h1, h2)

      def rz(j, oh_s):
        def body(l, _):
          rid = iota16 + l * 16
          d16 = dst_v[j, pl.ds(l * 16, 16)]
          dmod = jnp.bitwise_and(d16, 15)
          plsc.store_scatter(oh_s, [rid, dmod], zvec)
          return 0

        lax.fori_loop(0, SUB // 16, body, 0)

      pend = issue(0, slots[0])
      oh_pend = [None, None]
      sr_pend = None
      for j in range(nsub):
        if j + 1 < nsub:
          nxt = issue(j + 1, slots[(j + 1) % 2])
        rows_s, adrow_s, _, _ = slots[j % 2]
        oh_s, ddiv_s, sem_o = ohslots[j % 2]
        if oh_pend[j % 2] is not None:
          oh_pend[j % 2].wait()
          rz(j - 2, oh_s)
        pend[0].wait()
        pend[1].wait()

        def ohb(l, _):
          rid = iota16 + l * 16
          asg = plsc.load_gather(rows_s, [rid, c32])
          adg = plsc.load_gather(adrow_s, [rid, chead])
          t0 = asg + adg
          al = jnp.maximum(t0, 0.2 * t0)
          t1 = mvec + adg
          sh = jnp.maximum(t1, 0.2 * t1)
          exv = jnp.exp(al - sh)
          gid = iota16 + (base + j * SUB + l * 16)
          exv = jnp.where(gid < e_real, exv, 0.0)
          ex_v[pl.ds(l * 16, 16)] = exv
          d16 = dst_v[j, pl.ds(l * 16, 16)]
          dmod = jnp.bitwise_and(d16, 15)
          ddiv = jnp.right_shift(d16, 4)
          ddiv_s[0, pl.ds(l * 16, 16)] = ddiv
          plsc.store_scatter(oh_s, [rid, dmod], exv)
          return 0

        lax.fori_loop(0, SUB // 16, ohb, 0, unroll=2)

        if sr_pend is not None:
          sr_pend.wait()

        def scale(e2, _):
          exs = ex_v[pl.ds(e2, 16)][0]
          srows_v[e2, pl.ds(0, 16)] = rows_s[e2, pl.ds(0, 16)] * exs
          srows_v[e2, pl.ds(16, 16)] = rows_s[e2, pl.ds(16, 16)] * exs
          return 0

        lax.fori_loop(0, SUB, scale, 0, unroll=8)

        sr_pend = pltpu.async_copy(
            srows_v, out_sp.at[dst_v.at[j]], sem_sr, add=True)
        oh_pend[j % 2] = pltpu.async_copy(
            oh_s, den_sp.at[ddiv_s.at[0]], sem_o, add=True)
        if j + 1 < nsub:
          pend = nxt

      sr_pend.wait()
      for j in (nsub - 2, nsub - 1):
        q = j % 2
        if oh_pend[q] is not None:
          oh_pend[q].wait()
          rz(j, ohslots[q][0])
      return 0

    lax.fori_loop(0, n_chunks, chunk_body, 0)
    plsc.subcore_barrier()

    def drain(k, _):
      c = sid + 16 * k

      @pl.when(c < nch)
      def _():
        pltpu.sync_copy(out_sp.at[pl.ds(c * drn, drn)],
                        srows_v.at[pl.ds(0, drn)])
        pltpu.sync_copy(srows_v.at[pl.ds(0, drn)],
                        num_hbm.at[cid, pl.ds(c * drn, drn)])

      return 0

    lax.fori_loop(0, ndr, drain, 0)

    def draind(k, _):
      c = sid + 16 * k

      @pl.when(c < dch)
      def _():
        pltpu.sync_copy(den_sp.at[pl.ds(c * SUB, SUB)], oh_v)
        pltpu.sync_copy(oh_v, den_hbm.at[cid, pl.ds(c * SUB, SUB)])

      return 0

    lax.fori_loop(0, ndd, draind, 0)

  return sc_k(hsrc, srcp, dst3, adp, mrow)


def kernel(x, edge_index, batch, W_emb, b_emb, W1, att_src1, att_dst1, b1,
           W2, att_src2, att_dst2, b2, gamma1, beta1, gamma2, beta2,
           W_out, b_out):
  n = x.shape[0]
  e = edge_index.shape[1]
  g_groups = 64
  f32 = jnp.float32

  ep = ((e + 16 * CE - 1) // (16 * CE)) * (16 * CE)
  src = edge_index[0]
  dst = edge_index[1]
  srcp = jnp.concatenate([src, jnp.zeros((ep - e,), jnp.int32)])
  dstp = jnp.concatenate([dst, jnp.zeros((ep - e,), jnp.int32)])
  dst3 = dstp.reshape(16, (ep // 16) // 128, 128)
  batch2d = batch.reshape(n, 1)

  a1 = jnp.zeros((64, 2), f32)
  a1 = a1.at[0:32, 0].set(att_src1[0]).at[32:64, 1].set(att_src1[1])
  b1a = jnp.zeros((64, 2), f32)
  b1a = b1a.at[0:32, 0].set(att_dst1[0]).at[32:64, 1].set(att_dst1[1])
  a2 = att_src2.T
  b2a = att_dst2.T

  bemb2d = b_emb.reshape(1, 64)
  b1_2d = b1.reshape(1, 64)
  b2_2d = b2.reshape(1, 64)
  g1_2d = gamma1.reshape(1, 64)
  be1_2d = beta1.reshape(1, 64)
  g2_2d = gamma2.reshape(1, 64)
  be2_2d = beta2.reshape(1, 64)
  bout2d = b_out.reshape(1, 128)

  zpad15 = jnp.zeros((n, 15), f32)

  hw1, as1, ad1, m1 = _prep_call(True, x, W_emb, bemb2d, W1, a1, b1a, 2)
  hsrc1 = jnp.concatenate([
      jnp.concatenate([hw1[:, :32], as1[:, 0:1], zpad15], axis=1),
      jnp.concatenate([hw1[:, 32:], as1[:, 1:2], zpad15], axis=1),
  ], axis=0)
  adp1 = jnp.concatenate([ad1, jnp.zeros((n, 14), f32)], axis=1)
  mp1 = jnp.broadcast_to(m1.T, (2, 16))
  num1, den1 = _sc_edge_call(hsrc1, srcp, dst3, adp1, mp1,
                             n=n, e_real=e, head_is_core=True)
  den1f = jnp.stack(
      [den1[0].reshape(-1)[:n], den1[1].reshape(-1)[:n]], axis=1)
  conv1, s1, q1 = _post_call(2, num1, den1f, as1, ad1, m1, hw1, b1_2d)

  hw2, as2, ad2, m2 = _bnprep_call(conv1, s1, q1, g1_2d, be1_2d, W2, a2,
                                   b2a, 1)
  hsrc2 = jnp.concatenate([
      jnp.concatenate([hw2[:, :32], as2, zpad15], axis=1),
      jnp.concatenate([hw2[:, 32:], as2, zpad15], axis=1),
  ], axis=0)
  adp2 = jnp.concatenate([ad2, zpad15], axis=1)
  mp2 = jnp.broadcast_to(m2.T, (1, 16))
  num2, den2 = _sc_edge_call(hsrc2, srcp, dst3, adp2, mp2,
                             n=n, e_real=e, head_is_core=False)
  den2f = den2[0].reshape(-1)[:n].reshape(n, 1)
  conv2, s2, q2 = _post_call(1, num2, den2f, as2, ad2, m2, hw2, b2_2d)

  pmax, psum, pcnt = _pool_call(conv2, s2, q2, g2_2d, be2_2d, batch2d,
                                g_groups)
  return _final_call(pmax, psum, pcnt, W_out, b_out.reshape(1, 128))


# speedup vs baseline: 59.7047x; 1.1381x over previous
import functools

import jax
import jax.numpy as jnp
from jax import lax
from jax.experimental import pallas as pl
from jax.experimental.pallas import tpu as pltpu
from jax.experimental.pallas import tpu_sc as plsc

RB = 1000
CE = 1024
SUB = 128
NEG_INF = float("-inf")


def _elu(x):
  return jnp.where(x > 0, x, jnp.exp(jnp.minimum(x, 0.0)) - 1.0)


def _emit_planes(heads, hw, asb, adb, lo_o, hi_o, adp_o):
  z15 = jnp.zeros((hw.shape[0], 15), jnp.float32)
  as_hi = asb[:, 1:2] if heads > 1 else asb[:, 0:1]
  lo_o[...] = jnp.concatenate([hw[:, :32], asb[:, 0:1], z15], axis=1)
  hi_o[...] = jnp.concatenate([hw[:, 32:], as_hi, z15], axis=1)
  adp_o[...] = jnp.concatenate(
      [adb, jnp.zeros((hw.shape[0], 16 - heads), jnp.float32)], axis=1)


def _prep_kernel(embed, heads, h_r, wemb_r, bemb_r, w_r, a_r, b_r,
                 lo_o, hi_o, adp_o, m_o):
  i = pl.program_id(0)
  h = h_r[...]
  if embed:
    h = _elu(h @ wemb_r[...] + bemb_r[...])
  hw = h @ w_r[...]
  asb = hw @ a_r[...]
  adb = hw @ b_r[...]
  _emit_planes(heads, hw, asb, adb, lo_o, hi_o, adp_o)

  @pl.when(i == 0)
  def _():
    m_o[...] = jnp.full_like(m_o[...], NEG_INF)

  m_o[...] = jnp.maximum(m_o[...], jnp.max(asb, axis=0, keepdims=True))


def _plane_out_specs(n, heads):
  return (
      [
          pl.BlockSpec((RB, 48), lambda i: (i, 0)),
          pl.BlockSpec((RB, 48), lambda i: (i, 0)),
          pl.BlockSpec((RB, 16), lambda i: (i, 0)),
          pl.BlockSpec((1, heads), lambda i: (0, 0)),
      ],
      [
          jax.ShapeDtypeStruct((n, 48), jnp.float32),
          jax.ShapeDtypeStruct((n, 48), jnp.float32),
          jax.ShapeDtypeStruct((n, 16), jnp.float32),
          jax.ShapeDtypeStruct((1, heads), jnp.float32),
      ],
  )


def _prep_call(embed, h_in, wemb, bemb, w, a, b, heads):
  n = h_in.shape[0]
  nb = n // RB
  cin = h_in.shape[1]
  full = lambda shp: pl.BlockSpec(shp, lambda i: (0, 0))
  out_specs, out_shape = _plane_out_specs(n, heads)
  return pl.pallas_call(
      functools.partial(_prep_kernel, embed, heads),
      grid=(nb,),
      in_specs=[
          pl.BlockSpec((RB, cin), lambda i: (i, 0)),
          full(wemb.shape), full(bemb.shape), full(w.shape),
          full(a.shape), full(b.shape),
      ],
      out_specs=out_specs,
      out_shape=out_shape,
  )(h_in, wemb, bemb, w, a, b)


def _post_kernel(heads, n, num_lo_r, num_hi_r, den_r, lo_r, hi_r, adp_r,
                 m_r, b_r, conv_o, ssum_o, ssq_o):
  i = pl.program_id(0)
  c = 64 // heads
  lo = lo_r[...]
  hi = hi_r[...]
  hw = jnp.concatenate([lo[:, :32], hi[:, :32]], axis=1)
  if heads > 1:
    asb = jnp.concatenate([lo[:, 32:33], hi[:, 32:33]], axis=1)
  else:
    asb = lo[:, 32:33]
  adb = adp_r[...][:, :heads]
  m = m_r[...]
  t = asb + adb
  al = jnp.maximum(t, 0.2 * t)
  t2 = m + adb
  d2 = jnp.maximum(t2, 0.2 * t2)
  sex = jnp.exp(al - d2)
  den = den_r[...] + sex
  num = jnp.concatenate([num_lo_r[0], num_hi_r[0]], axis=1)
  parts = []
  for h in range(heads):
    nh = num[:, h * c:(h + 1) * c] + sex[:, h:h + 1] * hw[:, h * c:(h + 1) * c]
    parts.append(nh / (den[:, h:h + 1] + 1e-16))
  conv = (jnp.concatenate(parts, axis=1) if heads > 1 else parts[0]) + b_r[...]
  conv_o[...] = conv

  @pl.when(i == 0)
  def _():
    ssum_o[...] = jnp.zeros_like(ssum_o[...])
    ssq_o[...] = jnp.zeros_like(ssq_o[...])

  ssum_o[...] += jnp.sum(conv, axis=0, keepdims=True)
  ssq_o[...] += jnp.sum(conv * conv, axis=0, keepdims=True)


def _post_call(heads, num3, den, lo, hi, adp, m, bias):
  n = num3.shape[1]
  nb = n // RB
  full = lambda shp: pl.BlockSpec(shp, lambda i: (0, 0))
  return pl.pallas_call(
      functools.partial(_post_kernel, heads, n),
      grid=(nb,),
      in_specs=[
          pl.BlockSpec((1, RB, 32), lambda i: (0, i, 0)),
          pl.BlockSpec((1, RB, 32), lambda i: (1, i, 0)),
          pl.BlockSpec((RB, heads), lambda i: (i, 0)),
          pl.BlockSpec((RB, 48), lambda i: (i, 0)),
          pl.BlockSpec((RB, 48), lambda i: (i, 0)),
          pl.BlockSpec((RB, 16), lambda i: (i, 0)),
          full((1, heads)),
          full((1, 64)),
      ],
      out_specs=[
          pl.BlockSpec((RB, 64), lambda i: (i, 0)),
          full((1, 64)), full((1, 64)),
      ],
      out_shape=[
          jax.ShapeDtypeStruct((n, 64), jnp.float32),
          jax.ShapeDtypeStruct((1, 64), jnp.float32),
          jax.ShapeDtypeStruct((1, 64), jnp.float32),
      ],
  )(num3, num3, den, lo, hi, adp, m, bias)


def _bnprep_kernel(n, heads, conv_r, ssum_r, ssq_r, g_r, be_r, w_r, a_r,
                   b_r, lo_o, hi_o, adp_o, m_o):
  i = pl.program_id(0)
  mu = ssum_r[...] / n
  var = ssq_r[...] / n - mu * mu
  y = (conv_r[...] - mu) / jnp.sqrt(var + 1e-5) * g_r[...] + be_r[...]
  h = _elu(y)
  hw = h @ w_r[...]
  asb = hw @ a_r[...]
  adb = hw @ b_r[...]
  _emit_planes(heads, hw, asb, adb, lo_o, hi_o, adp_o)

  @pl.when(i == 0)
  def _():
    m_o[...] = jnp.full_like(m_o[...], NEG_INF)

  m_o[...] = jnp.maximum(m_o[...], jnp.max(asb, axis=0, keepdims=True))


def _bnprep_call(conv, ssum, ssq, gamma, beta, w, a, b, heads):
  n = conv.shape[0]
  nb = n // RB
  full = lambda shp: pl.BlockSpec(shp, lambda i: (0, 0))
  out_specs, out_shape = _plane_out_specs(n, heads)
  return pl.pallas_call(
      functools.partial(_bnprep_kernel, n, heads),
      grid=(nb,),
      in_specs=[
          pl.BlockSpec((RB, 64), lambda i: (i, 0)),
          full((1, 64)), full((1, 64)), full((1, 64)), full((1, 64)),
          full((64, 64)), full((64, heads)), full((64, heads)),
      ],
      out_specs=out_specs,
      out_shape=out_shape,
  )(conv, ssum, ssq, gamma, beta, w, a, b)


def _pool_kernel(n, g_groups, conv_r, ssum_r, ssq_r, g_r, be_r, bt_r,
                 pmax_o, psum_o, pcnt_o):
  i = pl.program_id(0)
  mu = ssum_r[...] / n
  var = ssq_r[...] / n - mu * mu
  y = (conv_r[...] - mu) / jnp.sqrt(var + 1e-5) * g_r[...] + be_r[...]
  h = _elu(y)
  bt = bt_r[...]

  @pl.when(i == 0)
  def _():
    pmax_o[...] = jnp.full_like(pmax_o[...], NEG_INF)
    psum_o[...] = jnp.zeros_like(psum_o[...])
    pcnt_o[...] = jnp.zeros_like(pcnt_o[...])

  g0 = bt[0, 0]
  g1 = bt[RB - 1, 0]

  def body(g, _):
    mask = bt == g
    hm = jnp.where(mask, h, NEG_INF)
    gmax = jnp.max(hm, axis=0, keepdims=True)
    hs = jnp.where(mask, h, 0.0)
    gsum = jnp.sum(hs, axis=0, keepdims=True)
    gcnt = jnp.sum(jnp.where(mask, 1.0, 0.0))
    pmax_o[pl.ds(g, 1), :] = jnp.maximum(pmax_o[pl.ds(g, 1), :], gmax)
    psum_o[pl.ds(g, 1), :] = psum_o[pl.ds(g, 1), :] + gsum
    pcnt_o[pl.ds(g, 1), :] = pcnt_o[pl.ds(g, 1), :] + gcnt
    return 0

  lax.fori_loop(g0, g1 + 1, body, 0)


def _pool_call(conv, ssum, ssq, gamma, beta, batch2d, g_groups):
  n = conv.shape[0]
  nb = n // RB
  full = lambda shp: pl.BlockSpec(shp, lambda i: (0, 0))
  return pl.pallas_call(
      functools.partial(_pool_kernel, n, g_groups),
      grid=(nb,),
      in_specs=[
          pl.BlockSpec((RB, 64), lambda i: (i, 0)),
          full((1, 64)), full((1, 64)), full((1, 64)), full((1, 64)),
          pl.BlockSpec((RB, 1), lambda i: (i, 0)),
      ],
      out_specs=[
          full((g_groups, 64)), full((g_groups, 64)), full((g_groups, 64)),
      ],
      out_shape=[
          jax.ShapeDtypeStruct((g_groups, 64), jnp.float32),
          jax.ShapeDtypeStruct((g_groups, 64), jnp.float32),
          jax.ShapeDtypeStruct((g_groups, 64), jnp.float32),
      ],
  )(conv, ssum, ssq, gamma, beta, batch2d)


def _final_kernel(pmax_r, psum_r, pcnt_r, w_r, b_r, out_o):
  pmax = pmax_r[...]
  xmax = jnp.where(pmax == NEG_INF, 0.0, pmax)
  xmean = psum_r[...] / jnp.maximum(pcnt_r[...], 1.0)
  comb = jnp.concatenate([xmax, xmean], axis=1)
  out_o[...] = comb @ w_r[...] + b_r[...]


def _final_call(pmax, psum, pcnt, w_out, b_out):
  g = pmax.shape[0]
  return pl.pallas_call(
      _final_kernel,
      out_shape=jax.ShapeDtypeStruct((g, 128), jnp.float32),
  )(pmax, psum, pcnt, w_out, b_out)


def _sc_edge_call(lo, hi, srcp, dst3, adp, mrow, *, n, e_real,
                  head_is_core):
  ep = srcp.shape[0]
  et = ep // 16
  n_chunks = et // CE
  nsub = CE // SUB
  drn = 80
  nch = n // drn
  ndr = (nch + 15) // 16
  nden = ((n // 16 + 127) // 128) * 128
  dch = nden // SUB
  ndd = (dch + 15) // 16
  mesh = plsc.VectorSubcoreMesh(core_axis_name="c", subcore_axis_name="s")

  @functools.partial(
      pl.kernel,
      mesh=mesh,
      compiler_params=pltpu.CompilerParams(
          needs_layout_passes=False, use_tc_tiling_on_sc=False),
      out_type=[
          jax.ShapeDtypeStruct((2, n, 32), jnp.float32),
          jax.ShapeDtypeStruct((2, nden, 16), jnp.float32),
      ],
      scratch_types=[
          pltpu.VMEM((CE,), jnp.int32),
          pltpu.VMEM((CE // 128, 128), jnp.int32),
          pltpu.VMEM((SUB + 16,), jnp.float32),
          pltpu.VMEM((SUB, 48), jnp.float32),
          pltpu.VMEM((SUB, 48), jnp.float32),
          pltpu.VMEM((SUB, 16), jnp.float32),
          pltpu.VMEM((SUB, 16), jnp.float32),
          pltpu.VMEM((SUB, 32), jnp.float32),
          pltpu.VMEM((SUB, 16), jnp.float32),
          pltpu.VMEM((SUB, 16), jnp.float32),
          pltpu.VMEM((1, 128), jnp.int32),
          pltpu.VMEM((1, 128), jnp.int32),
          pltpu.VMEM((16,), jnp.float32),
          pltpu.SemaphoreType.DMA,
          pltpu.SemaphoreType.DMA,
          pltpu.SemaphoreType.DMA,
          pltpu.SemaphoreType.DMA,
          pltpu.SemaphoreType.DMA,
          pltpu.SemaphoreType.DMA,
          pltpu.SemaphoreType.DMA,
          pltpu.VMEM_SHARED((n, 32), jnp.float32),
          pltpu.VMEM_SHARED((nden, 16), jnp.float32),
      ],
  )
  def sc_k(lo_hbm, hi_hbm, srcp_hbm, dst3_hbm, adp_hbm, m_hbm,
           num_hbm, den_hbm,
           src_v, dst_v, ex_v, rows_a, rows_b, adrow_a, adrow_b,
           srows_v, oh_a, oh_b, ddiv_a, ddiv_b, m_v,
           sem_ra, sem_rb, sem_aa, sem_ab, sem_oa, sem_ob, sem_sr,
           out_sp, den_sp):
    cid = lax.axis_index("c")
    sid = lax.axis_index("s")
    head = cid if head_is_core else 0
    pltpu.sync_copy(m_hbm.at[head], m_v)
    mvec = m_v[...]
    zvec = jnp.zeros((16,), jnp.float32)
    iota16 = lax.iota(jnp.int32, 16)
    c32 = jnp.full((16,), 32, jnp.int32)
    c0 = jnp.zeros((16,), jnp.int32)
    chead = c0 + head
    slots = ((rows_a, adrow_a, sem_ra, sem_aa),
             (rows_b, adrow_b, sem_rb, sem_ab))
    ohslots = ((oh_a, ddiv_a, sem_oa), (oh_b, ddiv_b, sem_ob))
    oh_v = oh_a

    def zs(r, _):
      srows_v[r, pl.ds(0, 16)] = zvec
      srows_v[r, pl.ds(16, 16)] = zvec
      oh_a[r, :] = zvec
      oh_b[r, :] = zvec
      return 0

    lax.fori_loop(0, SUB, zs, 0)

    def zcp(k, _):
      c = sid + 16 * k

      @pl.when(c < nch)
      def _():
        pltpu.sync_copy(srows_v.at[pl.ds(0, drn)],
                        out_sp.at[pl.ds(c * drn, drn)])

      return 0

    lax.fori_loop(0, ndr, zcp, 0)

    def zcd(k, _):
      c = sid + 16 * k

      @pl.when(c < dch)
      def _():
        pltpu.sync_copy(oh_v, den_sp.at[pl.ds(c * SUB, SUB)])

      return 0

    lax.fori_loop(0, ndd, zcd, 0)
    plsc.subcore_barrier()

    def chunk_body(ch, _):
      base = sid * et + ch * CE
      pltpu.sync_copy(srcp_hbm.at[pl.ds(base, CE)], src_v)
      pltpu.sync_copy(dst3_hbm.at[sid, pl.ds(ch * (CE // 128), CE // 128)],
                      dst_v)

      def issue(j, slot):
        rows_s, adrow_s, sem_r, sem_a = slot
        sl = src_v.at[pl.ds(j * SUB, SUB)]

        @pl.when(cid == 0)
        def _():
          pltpu.async_copy(lo_hbm.at[sl], rows_s, sem_r)

        @pl.when(cid == 1)
        def _():
          pltpu.async_copy(hi_hbm.at[sl], rows_s, sem_r)

        h1 = pltpu.make_async_copy(lo_hbm.at[sl], rows_s, sem_r)
        h2 = pltpu.async_copy(adp_hbm.at[dst_v.at[j]], adrow_s, sem_a)
        return (h1, h2)

      def rz(j, oh_s):
        def body(l, _):
          rid = iota16 + l * 16
          d16 = dst_v[j, pl.ds(l * 16, 16)]
          dmod = jnp.bitwise_and(d16, 15)
          plsc.store_scatter(oh_s, [rid, dmod], zvec)
          return 0

        lax.fori_loop(0, SUB // 16, body, 0)

      pend = issue(0, slots[0])
      oh_pend = [None, None]
      sr_pend = None
      for j in range(nsub):
        if j + 1 < nsub:
          nxt = issue(j + 1, slots[(j + 1) % 2])
        rows_s, adrow_s, _, _ = slots[j % 2]
        oh_s, ddiv_s, sem_o = ohslots[j % 2]
        if oh_pend[j % 2] is not None:
          oh_pend[j % 2].wait()
          rz(j - 2, oh_s)
        pend[0].wait()
        pend[1].wait()

        def ohb(l, _):
          rid = iota16 + l * 16
          asg = plsc.load_gather(rows_s, [rid, c32])
          adg = plsc.load_gather(adrow_s, [rid, chead])
          t0 = asg + adg
          al = jnp.maximum(t0, 0.2 * t0)
          t1 = mvec + adg
          sh = jnp.maximum(t1, 0.2 * t1)
          exv = jnp.exp(al - sh)
          gid = iota16 + (base + j * SUB + l * 16)
          exv = jnp.where(gid < e_real, exv, 0.0)
          ex_v[pl.ds(l * 16, 16)] = exv
          d16 = dst_v[j, pl.ds(l * 16, 16)]
          dmod = jnp.bitwise_and(d16, 15)
          ddiv = jnp.right_shift(d16, 4)
          ddiv_s[0, pl.ds(l * 16, 16)] = ddiv
          plsc.store_scatter(oh_s, [rid, dmod], exv)
          return 0

        lax.fori_loop(0, SUB // 16, ohb, 0, unroll=2)

        if sr_pend is not None:
          sr_pend.wait()

        def scale(e2, _):
          exs = ex_v[pl.ds(e2, 16)][0]
          srows_v[e2, pl.ds(0, 16)] = rows_s[e2, pl.ds(0, 16)] * exs
          srows_v[e2, pl.ds(16, 16)] = rows_s[e2, pl.ds(16, 16)] * exs
          return 0

        lax.fori_loop(0, SUB, scale, 0, unroll=8)

        sr_pend = pltpu.async_copy(
            srows_v, out_sp.at[dst_v.at[j]], sem_sr, add=True)
        oh_pend[j % 2] = pltpu.async_copy(
            oh_s, den_sp.at[ddiv_s.at[0]], sem_o, add=True)
        if j + 1 < nsub:
          pend = nxt

      sr_pend.wait()
      for j in (nsub - 2, nsub - 1):
        q = j % 2
        if oh_pend[q] is not None:
          oh_pend[q].wait()
          rz(j, ohslots[q][0])
      return 0

    lax.fori_loop(0, n_chunks, chunk_body, 0)
    plsc.subcore_barrier()

    def drain(k, _):
      c = sid + 16 * k

      @pl.when(c < nch)
      def _():
        pltpu.sync_copy(out_sp.at[pl.ds(c * drn, drn)],
                        srows_v.at[pl.ds(0, drn)])
        pltpu.sync_copy(srows_v.at[pl.ds(0, drn)],
                        num_hbm.at[cid, pl.ds(c * drn, drn)])

      return 0

    lax.fori_loop(0, ndr, drain, 0)

    def draind(k, _):
      c = sid + 16 * k

      @pl.when(c < dch)
      def _():
        pltpu.sync_copy(den_sp.at[pl.ds(c * SUB, SUB)], oh_v)
        pltpu.sync_copy(oh_v, den_hbm.at[cid, pl.ds(c * SUB, SUB)])

      return 0

    lax.fori_loop(0, ndd, draind, 0)

  return sc_k(lo, hi, srcp, dst3, adp, mrow)


def kernel(x, edge_index, batch, W_emb, b_emb, W1, att_src1, att_dst1, b1,
           W2, att_src2, att_dst2, b2, gamma1, beta1, gamma2, beta2,
           W_out, b_out):
  n = x.shape[0]
  e = edge_index.shape[1]
  g_groups = 64
  f32 = jnp.float32

  ep = ((e + 16 * CE - 1) // (16 * CE)) * (16 * CE)
  src = edge_index[0]
  dst = edge_index[1]
  srcp = jnp.concatenate([src, jnp.zeros((ep - e,), jnp.int32)])
  dstp = jnp.concatenate([dst, jnp.zeros((ep - e,), jnp.int32)])
  dst3 = dstp.reshape(16, (ep // 16) // 128, 128)
  batch2d = batch.reshape(n, 1)

  a1 = jnp.zeros((64, 2), f32)
  a1 = a1.at[0:32, 0].set(att_src1[0]).at[32:64, 1].set(att_src1[1])
  b1a = jnp.zeros((64, 2), f32)
  b1a = b1a.at[0:32, 0].set(att_dst1[0]).at[32:64, 1].set(att_dst1[1])
  a2 = att_src2.T
  b2a = att_dst2.T

  bemb2d = b_emb.reshape(1, 64)
  b1_2d = b1.reshape(1, 64)
  b2_2d = b2.reshape(1, 64)
  g1_2d = gamma1.reshape(1, 64)
  be1_2d = beta1.reshape(1, 64)
  g2_2d = gamma2.reshape(1, 64)
  be2_2d = beta2.reshape(1, 64)
  bout2d = b_out.reshape(1, 128)

  lo1, hi1, adp1, m1 = _prep_call(True, x, W_emb, bemb2d, W1, a1, b1a, 2)
  mp1 = jnp.broadcast_to(m1.T, (2, 16))
  num1, den1 = _sc_edge_call(lo1, hi1, srcp, dst3, adp1, mp1,
                             n=n, e_real=e, head_is_core=True)
  den1f = jnp.stack(
      [den1[0].reshape(-1)[:n], den1[1].reshape(-1)[:n]], axis=1)
  conv1, s1, q1 = _post_call(2, num1, den1f, lo1, hi1, adp1, m1, b1_2d)

  lo2, hi2, adp2, m2 = _bnprep_call(conv1, s1, q1, g1_2d, be1_2d, W2, a2,
                                    b2a, 1)
  mp2 = jnp.broadcast_to(m2.T, (1, 16))
  num2, den2 = _sc_edge_call(lo2, hi2, srcp, dst3, adp2, mp2,
                             n=n, e_real=e, head_is_core=False)
  den2f = den2[0].reshape(-1)[:n].reshape(n, 1)
  conv2, s2, q2 = _post_call(1, num2, den2f, lo2, hi2, adp2, m2, b2_2d)

  pmax, psum, pcnt = _pool_call(conv2, s2, q2, g2_2d, be2_2d, batch2d,
                                g_groups)
  return _final_call(pmax, psum, pcnt, W_out, b_out.reshape(1, 128))
